# Initial kernel scaffold; baseline (speedup 1.0000x reference)
import jax
import jax.numpy as jnp
from jax.experimental import pallas as pl


def kernel(x, edge_index, W0, b0, g0, beta0, W1, b1, g1, beta1, W2, b2, Wfc, bfc):
    raise NotImplementedError("write your pallas kernel here")


# speedup vs baseline: 11.0176x; 11.0176x over previous
import functools

import jax
import jax.numpy as jnp
from jax import lax
from jax.experimental import pallas as pl
from jax.experimental.pallas import tpu as pltpu
from jax.experimental.pallas import tpu_sc as plsc

NC = 2
NS = 16
NW = NC * NS
KCH = 80


def _sc_degree(dst, zeros1, n_pad):
    e = dst.shape[0]
    epw = e // NW
    nch = epw // KCH
    rpt = n_pad // NS
    mesh = plsc.VectorSubcoreMesh(core_axis_name="c", subcore_axis_name="s")

    @functools.partial(
        pl.kernel,
        mesh=mesh,
        out_type=jax.ShapeDtypeStruct((NC * n_pad,), jnp.float32),
        scratch_types=[
            pltpu.VMEM((KCH,), jnp.int32),
            pltpu.VMEM((KCH,), jnp.float32),
            pltpu.VMEM_SHARED((n_pad,), jnp.float32),
        ],
    )
    def deg_kernel(dst_hbm, z_hbm, out_hbm, idx_v, ones_v, acc_sh):
        c = lax.axis_index("c")
        s = lax.axis_index("s")
        wid = s * NC + c
        soff = pl.multiple_of(s * rpt, 128)
        for j in range(KCH // 16):
            ones_v[pl.ds(16 * j, 16)] = jnp.full((16,), 1.0, dtype=jnp.float32)
        pltpu.sync_copy(z_hbm.at[pl.ds(soff, rpt)],
                        acc_sh.at[pl.ds(soff, rpt)])
        plsc.subcore_barrier()
        base = wid * epw

        def body(i, carry):
            pltpu.sync_copy(dst_hbm.at[pl.ds(base + i * KCH, KCH)], idx_v)
            pltpu.sync_copy(ones_v, acc_sh.at[idx_v], add=True)
            return carry

        lax.fori_loop(0, nch, body, 0)
        plsc.subcore_barrier()
        ooff = pl.multiple_of(c * n_pad + s * rpt, 128)
        pltpu.sync_copy(acc_sh.at[pl.ds(soff, rpt)],
                        out_hbm.at[pl.ds(ooff, rpt)])

    return deg_kernel(dst, zeros1)


def _sc_propagate(feat, src, dst, zrows):
    n = feat.shape[0]
    e = src.shape[0]
    epw = e // NW
    nch = epw // KCH
    rpt = ((n + NS - 1) // NS + 7) // 8 * 8
    rlast = n - (NS - 1) * rpt
    mesh = plsc.VectorSubcoreMesh(core_axis_name="c", subcore_axis_name="s")

    @functools.partial(
        pl.kernel,
        mesh=mesh,
        out_type=jax.ShapeDtypeStruct((NC, n, 128), jnp.float32),
        scratch_types=[
            pltpu.VMEM((KCH,), jnp.int32),
            pltpu.VMEM((KCH,), jnp.int32),
            pltpu.VMEM((KCH, 128), jnp.float32),
            pltpu.VMEM_SHARED((n, 128), jnp.float32),
            pltpu.SemaphoreType.DMA,
        ],
    )
    def prop_kernel(f_hbm, src_hbm, dst_hbm, z_hbm, out_hbm,
                    si_v, di_v, rows_v, acc_sh, sem):
        c = lax.axis_index("c")
        s = lax.axis_index("s")
        wid = s * NC + c
        soff = pl.multiple_of(s * rpt, 8)

        def seed(rows):
            @pl.when(c == 0)
            def _():
                pltpu.sync_copy(f_hbm.at[pl.ds(soff, rows)],
                                acc_sh.at[pl.ds(soff, rows)])

            @pl.when(c != 0)
            def _():
                pltpu.sync_copy(z_hbm.at[pl.ds(soff, rows)],
                                acc_sh.at[pl.ds(soff, rows)])

        @pl.when(s < NS - 1)
        def _():
            seed(rpt)

        @pl.when(s == NS - 1)
        def _():
            seed(rlast)

        plsc.subcore_barrier()
        base = wid * epw

        def body(i, carry):
            off = base + i * KCH
            pltpu.sync_copy(src_hbm.at[pl.ds(off, KCH)], si_v)
            pltpu.sync_copy(dst_hbm.at[pl.ds(off, KCH)], di_v)
            pltpu.async_copy(f_hbm.at[si_v], rows_v, sem).wait()
            pltpu.sync_copy(rows_v, acc_sh.at[di_v], add=True)
            return carry

        lax.fori_loop(0, nch, body, 0)
        plsc.subcore_barrier()

        @pl.when(s < NS - 1)
        def _():
            pltpu.sync_copy(acc_sh.at[pl.ds(soff, rpt)],
                            out_hbm.at[c, pl.ds(soff, rpt)])

        @pl.when(s == NS - 1)
        def _():
            pltpu.sync_copy(acc_sh.at[pl.ds(soff, rlast)],
                            out_hbm.at[c, pl.ds(soff, rlast)])

    return prop_kernel(feat, src, dst, zrows)


_ROWS = 1000


def _tc_pre(degp, x, w0):
    n = x.shape[0]
    nb = n // _ROWS

    def body(degp_ref, x_ref, w_ref, f_ref, dinv_ref):
        dinv = lax.rsqrt(degp_ref[0] + degp_ref[1] + 1.0)
        dinv_ref[...] = dinv
        f_ref[...] = dinv * jnp.dot(x_ref[...], w_ref[...],
                                    preferred_element_type=jnp.float32)

    return pl.pallas_call(
        body,
        grid=(nb,),
        in_specs=[
            pl.BlockSpec((2, _ROWS, 1), lambda i: (0, i, 0)),
            pl.BlockSpec((_ROWS, 128), lambda i: (i, 0)),
            pl.BlockSpec((128, 128), lambda i: (0, 0)),
        ],
        out_specs=[
            pl.BlockSpec((_ROWS, 128), lambda i: (i, 0)),
            pl.BlockSpec((_ROWS, 1), lambda i: (i, 0)),
        ],
        out_shape=[
            jax.ShapeDtypeStruct((n, 128), jnp.float32),
            jax.ShapeDtypeStruct((n, 1), jnp.float32),
        ],
    )(degp, x, w0)


def _tc_mid(dinv, accp, gam, bet, wn):
    n = accp.shape[1]
    nb = n // _ROWS
    nf = float(n)

    def body(dinv_ref, acc_ref, gam_ref, bet_ref, w_ref, out_ref, stats):
        p = pl.program_id(0)
        z = dinv_ref[...] * (acc_ref[0] + acc_ref[1])

        @pl.when(p == 0)
        def _():
            @pl.when(pl.program_id(1) == 0)
            def _():
                stats[...] = jnp.zeros_like(stats)

            stats[0:1, :] = stats[0:1, :] + jnp.sum(z, axis=0, keepdims=True)
            stats[1:2, :] = stats[1:2, :] + jnp.sum(z * z, axis=0,
                                                    keepdims=True)

        @pl.when(p == 1)
        def _():
            m = stats[0:1, :] / nf
            var = stats[1:2, :] / nf - m * m
            rs = lax.rsqrt(var + 1e-5)
            h = jnp.maximum((z - m) * rs * gam_ref[...] + bet_ref[...], 0.0)
            out_ref[...] = dinv_ref[...] * jnp.dot(
                h, w_ref[...], preferred_element_type=jnp.float32)

    return pl.pallas_call(
        body,
        grid=(2, nb),
        in_specs=[
            pl.BlockSpec((_ROWS, 1), lambda p, i: (i, 0)),
            pl.BlockSpec((2, _ROWS, 128), lambda p, i: (0, i, 0)),
            pl.BlockSpec((1, 128), lambda p, i: (0, 0)),
            pl.BlockSpec((1, 128), lambda p, i: (0, 0)),
            pl.BlockSpec((128, 128), lambda p, i: (0, 0)),
        ],
        out_specs=pl.BlockSpec((_ROWS, 128), lambda p, i: (i, 0)),
        out_shape=jax.ShapeDtypeStruct((n, 128), jnp.float32),
        scratch_shapes=[pltpu.VMEM((2, 128), jnp.float32)],
    )(dinv, accp, gam, bet, wn)


def _tc_final(dinv, accp, b2, wfc, bfc):
    n = accp.shape[1]
    c = wfc.shape[1]
    nb = n // _ROWS

    def body(dinv_ref, acc_ref, b_ref, w_ref, bf_ref, out_ref):
        z = dinv_ref[...] * (acc_ref[0] + acc_ref[1]) + b_ref[...]
        out_ref[...] = jnp.dot(z, w_ref[...],
                               preferred_element_type=jnp.float32) + bf_ref[...]

    return pl.pallas_call(
        body,
        grid=(nb,),
        in_specs=[
            pl.BlockSpec((_ROWS, 1), lambda i: (i, 0)),
            pl.BlockSpec((2, _ROWS, 128), lambda i: (0, i, 0)),
            pl.BlockSpec((1, 128), lambda i: (0, 0)),
            pl.BlockSpec((128, c), lambda i: (0, 0)),
            pl.BlockSpec((1, c), lambda i: (0, 0)),
        ],
        out_specs=pl.BlockSpec((_ROWS, c), lambda i: (i, 0)),
        out_shape=jax.ShapeDtypeStruct((n, c), jnp.float32),
    )(dinv, accp, b2, wfc, bfc)


def kernel(x, edge_index, W0, b0, g0, beta0, W1, b1, g1, beta1, W2, b2,
           Wfc, bfc):
    n = x.shape[0]
    src = edge_index[0]
    dst = edge_index[1]
    n_pad = ((n + NS * 128 - 1) // (NS * 128)) * (NS * 128)
    zeros1 = jnp.zeros((n_pad,), jnp.float32)
    zrows = jnp.zeros((n, 128), jnp.float32)

    degp = _sc_degree(dst, zeros1, n_pad).reshape(2, n_pad)[:, :n]
    degp = degp.reshape(2, n, 1)
    feat0, dinv = _tc_pre(degp, x, W0)
    acc0 = _sc_propagate(feat0, src, dst, zrows)
    feat1 = _tc_mid(dinv, acc0, g0.reshape(1, 128), beta0.reshape(1, 128), W1)
    acc1 = _sc_propagate(feat1, src, dst, zrows)
    feat2 = _tc_mid(dinv, acc1, g1.reshape(1, 128), beta1.reshape(1, 128), W2)
    acc2 = _sc_propagate(feat2, src, dst, zrows)
    return _tc_final(dinv, acc2, b2.reshape(1, 128), Wfc,
                     bfc.reshape(1, bfc.shape[0]))


# speedup vs baseline: 14.5913x; 1.3244x over previous
import functools

import jax
import jax.numpy as jnp
from jax import lax
from jax.experimental import pallas as pl
from jax.experimental.pallas import tpu as pltpu
from jax.experimental.pallas import tpu_sc as plsc

NC = 2
NS = 16
NW = NC * NS
KCH = 16


def _sc_degree(dst, zeros1, n_pad):
    e = dst.shape[0]
    epw = e // NW
    nch = epw // KCH
    rpt = n_pad // NS
    mesh = plsc.VectorSubcoreMesh(core_axis_name="c", subcore_axis_name="s")

    @functools.partial(
        pl.kernel,
        mesh=mesh,
        out_type=jax.ShapeDtypeStruct((NC * n_pad,), jnp.float32),
        scratch_types=[
            pltpu.VMEM((epw,), jnp.int32),
            pltpu.VMEM((KCH,), jnp.float32),
            pltpu.VMEM_SHARED((n_pad,), jnp.float32),
            pltpu.SemaphoreType.DMA,
        ],
    )
    def deg_kernel(dst_hbm, z_hbm, out_hbm, di_all, ones_v, acc_sh, sem):
        c = lax.axis_index("c")
        s = lax.axis_index("s")
        wid = s * NC + c
        soff = pl.multiple_of(s * rpt, 128)
        ones_v[...] = jnp.full((KCH,), 1.0, dtype=jnp.float32)
        pltpu.sync_copy(
            dst_hbm.at[pl.ds(pl.multiple_of(wid * epw, 8), epw)], di_all)
        pltpu.sync_copy(z_hbm.at[pl.ds(soff, rpt)],
                        acc_sh.at[pl.ds(soff, rpt)])
        plsc.subcore_barrier()

        def fire(i, carry):
            di = di_all[pl.ds(i * KCH, KCH)]
            pltpu.async_copy(ones_v, acc_sh.at[di], sem, add=True)
            return carry

        lax.fori_loop(0, nch, fire, 0)

        def drain(i, carry):
            di = di_all[pl.ds(i * KCH, KCH)]
            pltpu.make_async_copy(ones_v, acc_sh.at[di], sem).wait()
            return carry

        lax.fori_loop(0, nch, drain, 0)
        plsc.subcore_barrier()
        ooff = pl.multiple_of(c * n_pad + s * rpt, 128)
        pltpu.sync_copy(acc_sh.at[pl.ds(soff, rpt)],
                        out_hbm.at[pl.ds(ooff, rpt)])

    return deg_kernel(dst, zeros1)


_NBUF = 5
_LA = 2


def _sc_propagate(feat, src, dst, zrows):
    n = feat.shape[0]
    e = src.shape[0]
    epw = e // NW
    nch = epw // KCH
    assert nch % _NBUF == 0 and nch // _NBUF >= 3
    rpt = ((n + NS - 1) // NS + 7) // 8 * 8
    rlast = n - (NS - 1) * rpt
    mesh = plsc.VectorSubcoreMesh(core_axis_name="c", subcore_axis_name="s")

    @functools.partial(
        pl.kernel,
        mesh=mesh,
        out_type=jax.ShapeDtypeStruct((NC, n, 128), jnp.float32),
        scratch_types=[
            pltpu.VMEM((epw,), jnp.int32),
            pltpu.VMEM((epw,), jnp.int32),
            pltpu.VMEM((_NBUF, KCH, 128), jnp.float32),
            pltpu.VMEM_SHARED((n, 128), jnp.float32),
        ] + [pltpu.SemaphoreType.DMA] * (2 * _NBUF),
    )
    def prop_kernel(f_hbm, src_hbm, dst_hbm, z_hbm, out_hbm,
                    si_all, di_all, rows, acc_sh, *sems):
        sem_g = sems[:_NBUF]
        sem_s = sems[_NBUF:]
        c = lax.axis_index("c")
        s = lax.axis_index("s")
        wid = s * NC + c
        soff = pl.multiple_of(s * rpt, 8)
        ebase = pl.multiple_of(wid * epw, 8)

        pltpu.sync_copy(src_hbm.at[pl.ds(ebase, epw)], si_all)
        pltpu.sync_copy(dst_hbm.at[pl.ds(ebase, epw)], di_all)

        def seed(nrows):
            @pl.when(c == 0)
            def _():
                pltpu.sync_copy(f_hbm.at[pl.ds(soff, nrows)],
                                acc_sh.at[pl.ds(soff, nrows)])

            @pl.when(c != 0)
            def _():
                pltpu.sync_copy(z_hbm.at[pl.ds(soff, nrows)],
                                acc_sh.at[pl.ds(soff, nrows)])

        @pl.when(s < NS - 1)
        def _():
            seed(rpt)

        @pl.when(s == NS - 1)
        def _():
            seed(rlast)

        plsc.subcore_barrier()

        def start_gather(i, b):
            si = si_all[pl.ds(i * KCH, KCH)]
            pltpu.async_copy(f_hbm.at[si], rows.at[b], sem_g[b])

        def wait_gather(i, b):
            si = si_all[pl.ds(i * KCH, KCH)]
            pltpu.make_async_copy(f_hbm.at[si], rows.at[b], sem_g[b]).wait()

        def start_scatter(i, b):
            di = di_all[pl.ds(i * KCH, KCH)]
            pltpu.async_copy(rows.at[b], acc_sh.at[di], sem_s[b], add=True)

        def wait_scatter(i, b):
            di = di_all[pl.ds(i * KCH, KCH)]
            pltpu.make_async_copy(rows.at[b], acc_sh.at[di], sem_s[b]).wait()

        for i in range(_LA):
            start_gather(i, i)

        def step(i, b, bk, first):
            wait_gather(i, b)
            start_scatter(i, b)
            k = i + _LA
            if not first:
                wait_scatter(k - _NBUF, bk)
            start_gather(k, bk)

        for b in range(_NBUF):
            k = b + _LA
            if k < _NBUF:
                step(b, b, k, first=True)
            else:
                step(b, b, k - _NBUF, first=False)

        def group(jo, carry):
            j = jo * _NBUF
            for b in range(_NBUF):
                i = j + b
                step(i, b, (b + _LA) % _NBUF, first=False)
            return carry

        lax.fori_loop(1, nch // _NBUF - 1, group, 0)

        jl = nch - _NBUF
        for b in range(_NBUF):
            i = jl + b
            k = i + _LA
            wait_gather(i, b)
            start_scatter(i, b)
            if k < nch:
                bk = (b + _LA) % _NBUF
                wait_scatter(k - _NBUF, bk)
                start_gather(k, bk)

        for b in range(_NBUF):
            wait_scatter(jl + b, b)

        plsc.subcore_barrier()

        @pl.when(s < NS - 1)
        def _():
            pltpu.sync_copy(acc_sh.at[pl.ds(soff, rpt)],
                            out_hbm.at[c, pl.ds(soff, rpt)])

        @pl.when(s == NS - 1)
        def _():
            pltpu.sync_copy(acc_sh.at[pl.ds(soff, rlast)],
                            out_hbm.at[c, pl.ds(soff, rlast)])

    return prop_kernel(feat, src, dst, zrows)

    return prop_kernel(feat, src, dst, zrows)


_ROWS = 1000


def _tc_pre(degp, x, w0):
    n = x.shape[0]
    nb = n // _ROWS

    def body(degp_ref, x_ref, w_ref, f_ref, dinv_ref):
        dinv = lax.rsqrt(degp_ref[0] + degp_ref[1] + 1.0)
        dinv_ref[...] = dinv
        f_ref[...] = dinv * jnp.dot(x_ref[...], w_ref[...],
                                    preferred_element_type=jnp.float32)

    return pl.pallas_call(
        body,
        grid=(nb,),
        in_specs=[
            pl.BlockSpec((2, _ROWS, 1), lambda i: (0, i, 0)),
            pl.BlockSpec((_ROWS, 128), lambda i: (i, 0)),
            pl.BlockSpec((128, 128), lambda i: (0, 0)),
        ],
        out_specs=[
            pl.BlockSpec((_ROWS, 128), lambda i: (i, 0)),
            pl.BlockSpec((_ROWS, 1), lambda i: (i, 0)),
        ],
        out_shape=[
            jax.ShapeDtypeStruct((n, 128), jnp.float32),
            jax.ShapeDtypeStruct((n, 1), jnp.float32),
        ],
    )(degp, x, w0)


def _tc_mid(dinv, accp, gam, bet, wn):
    n = accp.shape[1]
    nb = n // _ROWS
    nf = float(n)

    def body(dinv_ref, acc_ref, gam_ref, bet_ref, w_ref, out_ref, stats):
        p = pl.program_id(0)
        z = dinv_ref[...] * (acc_ref[0] + acc_ref[1])

        @pl.when(p == 0)
        def _():
            @pl.when(pl.program_id(1) == 0)
            def _():
                stats[...] = jnp.zeros_like(stats)

            stats[0:1, :] = stats[0:1, :] + jnp.sum(z, axis=0, keepdims=True)
            stats[1:2, :] = stats[1:2, :] + jnp.sum(z * z, axis=0,
                                                    keepdims=True)

        @pl.when(p == 1)
        def _():
            m = stats[0:1, :] / nf
            var = stats[1:2, :] / nf - m * m
            rs = lax.rsqrt(var + 1e-5)
            h = jnp.maximum((z - m) * rs * gam_ref[...] + bet_ref[...], 0.0)
            out_ref[...] = dinv_ref[...] * jnp.dot(
                h, w_ref[...], preferred_element_type=jnp.float32)

    return pl.pallas_call(
        body,
        grid=(2, nb),
        in_specs=[
            pl.BlockSpec((_ROWS, 1), lambda p, i: (i, 0)),
            pl.BlockSpec((2, _ROWS, 128), lambda p, i: (0, i, 0)),
            pl.BlockSpec((1, 128), lambda p, i: (0, 0)),
            pl.BlockSpec((1, 128), lambda p, i: (0, 0)),
            pl.BlockSpec((128, 128), lambda p, i: (0, 0)),
        ],
        out_specs=pl.BlockSpec((_ROWS, 128), lambda p, i: (i, 0)),
        out_shape=jax.ShapeDtypeStruct((n, 128), jnp.float32),
        scratch_shapes=[pltpu.VMEM((2, 128), jnp.float32)],
    )(dinv, accp, gam, bet, wn)


def _tc_final(dinv, accp, b2, wfc, bfc):
    n = accp.shape[1]
    c = wfc.shape[1]
    nb = n // _ROWS

    def body(dinv_ref, acc_ref, b_ref, w_ref, bf_ref, out_ref):
        z = dinv_ref[...] * (acc_ref[0] + acc_ref[1]) + b_ref[...]
        out_ref[...] = jnp.dot(z, w_ref[...],
                               preferred_element_type=jnp.float32) + bf_ref[...]

    return pl.pallas_call(
        body,
        grid=(nb,),
        in_specs=[
            pl.BlockSpec((_ROWS, 1), lambda i: (i, 0)),
            pl.BlockSpec((2, _ROWS, 128), lambda i: (0, i, 0)),
            pl.BlockSpec((1, 128), lambda i: (0, 0)),
            pl.BlockSpec((128, c), lambda i: (0, 0)),
            pl.BlockSpec((1, c), lambda i: (0, 0)),
        ],
        out_specs=pl.BlockSpec((_ROWS, c), lambda i: (i, 0)),
        out_shape=jax.ShapeDtypeStruct((n, c), jnp.float32),
    )(dinv, accp, b2, wfc, bfc)


def kernel(x, edge_index, W0, b0, g0, beta0, W1, b1, g1, beta1, W2, b2,
           Wfc, bfc):
    n = x.shape[0]
    src = edge_index[0]
    dst = edge_index[1]
    n_pad = ((n + NS * 128 - 1) // (NS * 128)) * (NS * 128)
    zeros1 = jnp.zeros((n_pad,), jnp.float32)
    zrows = jnp.zeros((n, 128), jnp.float32)

    degp = _sc_degree(dst, zeros1, n_pad).reshape(2, n_pad)[:, :n]
    degp = degp.reshape(2, n, 1)
    feat0, dinv = _tc_pre(degp, x, W0)
    acc0 = _sc_propagate(feat0, src, dst, zrows)
    feat1 = _tc_mid(dinv, acc0, g0.reshape(1, 128), beta0.reshape(1, 128), W1)
    acc1 = _sc_propagate(feat1, src, dst, zrows)
    feat2 = _tc_mid(dinv, acc1, g1.reshape(1, 128), beta1.reshape(1, 128), W2)
    acc2 = _sc_propagate(feat2, src, dst, zrows)
    return _tc_final(dinv, acc2, b2.reshape(1, 128), Wfc,
                     bfc.reshape(1, bfc.shape[0]))


# speedup vs baseline: 19.2862x; 1.3218x over previous
import functools

import jax
import jax.numpy as jnp
from jax import lax
from jax.experimental import pallas as pl
from jax.experimental.pallas import tpu as pltpu
from jax.experimental.pallas import tpu_sc as plsc

NC = 2
NS = 16
NW = NC * NS
KCH = 16


def _sc_degree(dst, zeros1, n_pad):
    e = dst.shape[0]
    epw = e // NW
    nch = epw // KCH
    rpt = n_pad // NS
    mesh = plsc.VectorSubcoreMesh(core_axis_name="c", subcore_axis_name="s")

    @functools.partial(
        pl.kernel,
        mesh=mesh,
        out_type=jax.ShapeDtypeStruct((NC * n_pad,), jnp.float32),
        scratch_types=[
            pltpu.VMEM((epw,), jnp.int32),
            pltpu.VMEM((KCH,), jnp.float32),
            pltpu.VMEM_SHARED((n_pad,), jnp.float32),
            pltpu.SemaphoreType.DMA,
        ],
    )
    def deg_kernel(dst_hbm, z_hbm, out_hbm, di_all, ones_v, acc_sh, sem):
        c = lax.axis_index("c")
        s = lax.axis_index("s")
        wid = s * NC + c
        soff = pl.multiple_of(s * rpt, 128)
        ones_v[...] = jnp.full((KCH,), 1.0, dtype=jnp.float32)
        pltpu.sync_copy(
            dst_hbm.at[pl.ds(pl.multiple_of(wid * epw, 8), epw)], di_all)
        pltpu.sync_copy(z_hbm.at[pl.ds(soff, rpt)],
                        acc_sh.at[pl.ds(soff, rpt)])
        plsc.subcore_barrier()

        def fire(i, carry):
            di = di_all[pl.ds(i * KCH, KCH)]
            pltpu.async_copy(ones_v, acc_sh.at[di], sem, add=True)
            return carry

        lax.fori_loop(0, nch, fire, 0)

        def drain(i, carry):
            di = di_all[pl.ds(i * KCH, KCH)]
            pltpu.make_async_copy(ones_v, acc_sh.at[di], sem).wait()
            return carry

        lax.fori_loop(0, nch, drain, 0)
        plsc.subcore_barrier()
        ooff = pl.multiple_of(c * n_pad + s * rpt, 128)
        pltpu.sync_copy(acc_sh.at[pl.ds(soff, rpt)],
                        out_hbm.at[pl.ds(ooff, rpt)])

    return deg_kernel(dst, zeros1)


_NBUF = 5
_LA = 3


def _sc_propagate(feat, src, dst, zrows):
    n = feat.shape[0]
    e = src.shape[0]
    epw = e // NW
    nch = epw // KCH
    assert nch % _NBUF == 0 and nch // _NBUF >= 3
    rpt = ((n + NS - 1) // NS + 7) // 8 * 8
    rlast = n - (NS - 1) * rpt
    mesh = plsc.VectorSubcoreMesh(core_axis_name="c", subcore_axis_name="s")

    @functools.partial(
        pl.kernel,
        mesh=mesh,
        out_type=jax.ShapeDtypeStruct((NC, n, 128), jnp.float32),
        scratch_types=[
            pltpu.VMEM((epw,), jnp.int32),
            pltpu.VMEM((epw,), jnp.int32),
            pltpu.VMEM((_NBUF, KCH, 128), jnp.float32),
            pltpu.VMEM_SHARED((n, 128), jnp.float32),
        ] + [pltpu.SemaphoreType.DMA] * (2 * _NBUF),
    )
    def prop_kernel(f_hbm, src_hbm, dst_hbm, z_hbm, out_hbm,
                    si_all, di_all, rows, acc_sh, *sems):
        sem_g = sems[:_NBUF]
        sem_s = sems[_NBUF:]
        c = lax.axis_index("c")
        s = lax.axis_index("s")
        wid = s * NC + c
        soff = pl.multiple_of(s * rpt, 8)
        ebase = pl.multiple_of(wid * epw, 8)

        pltpu.sync_copy(src_hbm.at[pl.ds(ebase, epw)], si_all)
        pltpu.sync_copy(dst_hbm.at[pl.ds(ebase, epw)], di_all)

        def seed(nrows):
            @pl.when(c == 0)
            def _():
                pltpu.sync_copy(f_hbm.at[pl.ds(soff, nrows)],
                                acc_sh.at[pl.ds(soff, nrows)])

            @pl.when(c != 0)
            def _():
                pltpu.sync_copy(z_hbm.at[pl.ds(soff, nrows)],
                                acc_sh.at[pl.ds(soff, nrows)])

        @pl.when(s < NS - 1)
        def _():
            seed(rpt)

        @pl.when(s == NS - 1)
        def _():
            seed(rlast)

        plsc.subcore_barrier()

        def start_gather(i, b):
            si = si_all[pl.ds(i * KCH, KCH)]
            pltpu.async_copy(f_hbm.at[si], rows.at[b], sem_g[b])

        def wait_gather(i, b):
            si = si_all[pl.ds(i * KCH, KCH)]
            pltpu.make_async_copy(f_hbm.at[si], rows.at[b], sem_g[b]).wait()

        def start_scatter(i, b):
            di = di_all[pl.ds(i * KCH, KCH)]
            pltpu.async_copy(rows.at[b], acc_sh.at[di], sem_s[b], add=True)

        def wait_scatter(i, b):
            di = di_all[pl.ds(i * KCH, KCH)]
            pltpu.make_async_copy(rows.at[b], acc_sh.at[di], sem_s[b]).wait()

        for i in range(_LA):
            start_gather(i, i)

        def step(i, b, bk, first):
            wait_gather(i, b)
            start_scatter(i, b)
            k = i + _LA
            if not first:
                wait_scatter(k - _NBUF, bk)
            start_gather(k, bk)

        for b in range(_NBUF):
            k = b + _LA
            if k < _NBUF:
                step(b, b, k, first=True)
            else:
                step(b, b, k - _NBUF, first=False)

        def group(jo, carry):
            j = jo * _NBUF
            for b in range(_NBUF):
                i = j + b
                step(i, b, (b + _LA) % _NBUF, first=False)
            return carry

        lax.fori_loop(1, nch // _NBUF - 1, group, 0)

        jl = nch - _NBUF
        for b in range(_NBUF):
            i = jl + b
            k = i + _LA
            wait_gather(i, b)
            start_scatter(i, b)
            if k < nch:
                bk = (b + _LA) % _NBUF
                wait_scatter(k - _NBUF, bk)
                start_gather(k, bk)

        for b in range(_NBUF):
            wait_scatter(jl + b, b)

        plsc.subcore_barrier()

        @pl.when(s < NS - 1)
        def _():
            pltpu.sync_copy(acc_sh.at[pl.ds(soff, rpt)],
                            out_hbm.at[c, pl.ds(soff, rpt)])

        @pl.when(s == NS - 1)
        def _():
            pltpu.sync_copy(acc_sh.at[pl.ds(soff, rlast)],
                            out_hbm.at[c, pl.ds(soff, rlast)])

    return prop_kernel(feat, src, dst, zrows)

    return prop_kernel(feat, src, dst, zrows)


_ROWS = 1000


def _tc_pre(degp, x, w0):
    n = x.shape[0]
    nb = n // _ROWS

    def body(degp_ref, x_ref, w_ref, f_ref, dinv_ref):
        dinv = lax.rsqrt(degp_ref[0] + degp_ref[1] + 1.0)
        dinv_ref[...] = dinv
        f_ref[...] = dinv * jnp.dot(x_ref[...], w_ref[...],
                                    preferred_element_type=jnp.float32)

    return pl.pallas_call(
        body,
        grid=(nb,),
        in_specs=[
            pl.BlockSpec((2, _ROWS, 1), lambda i: (0, i, 0)),
            pl.BlockSpec((_ROWS, 128), lambda i: (i, 0)),
            pl.BlockSpec((128, 128), lambda i: (0, 0)),
        ],
        out_specs=[
            pl.BlockSpec((_ROWS, 128), lambda i: (i, 0)),
            pl.BlockSpec((_ROWS, 1), lambda i: (i, 0)),
        ],
        out_shape=[
            jax.ShapeDtypeStruct((n, 128), jnp.float32),
            jax.ShapeDtypeStruct((n, 1), jnp.float32),
        ],
    )(degp, x, w0)


def _tc_mid(dinv, accp, gam, bet, wn):
    n = accp.shape[1]
    nb = n // _ROWS
    nf = float(n)

    def body(dinv_ref, acc_ref, gam_ref, bet_ref, w_ref, out_ref, stats):
        p = pl.program_id(0)
        z = dinv_ref[...] * (acc_ref[0] + acc_ref[1])

        @pl.when(p == 0)
        def _():
            @pl.when(pl.program_id(1) == 0)
            def _():
                stats[...] = jnp.zeros_like(stats)

            stats[0:1, :] = stats[0:1, :] + jnp.sum(z, axis=0, keepdims=True)
            stats[1:2, :] = stats[1:2, :] + jnp.sum(z * z, axis=0,
                                                    keepdims=True)

        @pl.when(p == 1)
        def _():
            m = stats[0:1, :] / nf
            var = stats[1:2, :] / nf - m * m
            rs = lax.rsqrt(var + 1e-5)
            h = jnp.maximum((z - m) * rs * gam_ref[...] + bet_ref[...], 0.0)
            out_ref[...] = dinv_ref[...] * jnp.dot(
                h, w_ref[...], preferred_element_type=jnp.float32)

    return pl.pallas_call(
        body,
        grid=(2, nb),
        in_specs=[
            pl.BlockSpec((_ROWS, 1), lambda p, i: (i, 0)),
            pl.BlockSpec((2, _ROWS, 128), lambda p, i: (0, i, 0)),
            pl.BlockSpec((1, 128), lambda p, i: (0, 0)),
            pl.BlockSpec((1, 128), lambda p, i: (0, 0)),
            pl.BlockSpec((128, 128), lambda p, i: (0, 0)),
        ],
        out_specs=pl.BlockSpec((_ROWS, 128), lambda p, i: (i, 0)),
        out_shape=jax.ShapeDtypeStruct((n, 128), jnp.float32),
        scratch_shapes=[pltpu.VMEM((2, 128), jnp.float32)],
    )(dinv, accp, gam, bet, wn)


def _tc_final(dinv, accp, b2, wfc, bfc):
    n = accp.shape[1]
    c = wfc.shape[1]
    nb = n // _ROWS

    def body(dinv_ref, acc_ref, b_ref, w_ref, bf_ref, out_ref):
        z = dinv_ref[...] * (acc_ref[0] + acc_ref[1]) + b_ref[...]
        out_ref[...] = jnp.dot(z, w_ref[...],
                               preferred_element_type=jnp.float32) + bf_ref[...]

    return pl.pallas_call(
        body,
        grid=(nb,),
        in_specs=[
            pl.BlockSpec((_ROWS, 1), lambda i: (i, 0)),
            pl.BlockSpec((2, _ROWS, 128), lambda i: (0, i, 0)),
            pl.BlockSpec((1, 128), lambda i: (0, 0)),
            pl.BlockSpec((128, c), lambda i: (0, 0)),
            pl.BlockSpec((1, c), lambda i: (0, 0)),
        ],
        out_specs=pl.BlockSpec((_ROWS, c), lambda i: (i, 0)),
        out_shape=jax.ShapeDtypeStruct((n, c), jnp.float32),
    )(dinv, accp, b2, wfc, bfc)


def kernel(x, edge_index, W0, b0, g0, beta0, W1, b1, g1, beta1, W2, b2,
           Wfc, bfc):
    n = x.shape[0]
    src = edge_index[0]
    dst = edge_index[1]
    n_pad = ((n + NS * 128 - 1) // (NS * 128)) * (NS * 128)
    zeros1 = jnp.zeros((n_pad,), jnp.float32)
    zrows = jnp.zeros((n, 128), jnp.float32)

    degp = _sc_degree(dst, zeros1, n_pad).reshape(2, n_pad)[:, :n]
    degp = degp.reshape(2, n, 1)
    feat0, dinv = _tc_pre(degp, x, W0)
    acc0 = _sc_propagate(feat0, src, dst, zrows)
    feat1 = _tc_mid(dinv, acc0, g0.reshape(1, 128), beta0.reshape(1, 128), W1)
    acc1 = _sc_propagate(feat1, src, dst, zrows)
    feat2 = _tc_mid(dinv, acc1, g1.reshape(1, 128), beta1.reshape(1, 128), W2)
    acc2 = _sc_propagate(feat2, src, dst, zrows)
    return _tc_final(dinv, acc2, b2.reshape(1, 128), Wfc,
                     bfc.reshape(1, bfc.shape[0]))


# speedup vs baseline: 22.6258x; 1.1732x over previous
import functools

import jax
import jax.numpy as jnp
from jax import lax
from jax.experimental import pallas as pl
from jax.experimental.pallas import tpu as pltpu
from jax.experimental.pallas import tpu_sc as plsc

NC = 2
NS = 16
NW = NC * NS
KCH = 16


def _sc_degree(dst, zeros1, n_pad):
    e = dst.shape[0]
    epw = e // NW
    nch = epw // KCH
    rpt = n_pad // NS
    mesh = plsc.VectorSubcoreMesh(core_axis_name="c", subcore_axis_name="s")

    @functools.partial(
        pl.kernel,
        mesh=mesh,
        out_type=jax.ShapeDtypeStruct((NC * n_pad,), jnp.float32),
        scratch_types=[
            pltpu.VMEM((epw,), jnp.int32),
            pltpu.VMEM((KCH,), jnp.float32),
            pltpu.VMEM_SHARED((n_pad,), jnp.float32),
            pltpu.SemaphoreType.DMA,
        ],
    )
    def deg_kernel(dst_hbm, z_hbm, out_hbm, di_all, ones_v, acc_sh, sem):
        c = lax.axis_index("c")
        s = lax.axis_index("s")
        wid = s * NC + c
        soff = pl.multiple_of(s * rpt, 128)
        ones_v[...] = jnp.full((KCH,), 1.0, dtype=jnp.float32)
        pltpu.sync_copy(
            dst_hbm.at[pl.ds(pl.multiple_of(wid * epw, 8), epw)], di_all)
        pltpu.sync_copy(z_hbm.at[pl.ds(soff, rpt)],
                        acc_sh.at[pl.ds(soff, rpt)])
        plsc.subcore_barrier()

        def fire(i, carry):
            di = di_all[pl.ds(i * KCH, KCH)]
            pltpu.async_copy(ones_v, acc_sh.at[di], sem, add=True)
            return carry

        lax.fori_loop(0, nch, fire, 0)

        def drain(i, carry):
            di = di_all[pl.ds(i * KCH, KCH)]
            pltpu.make_async_copy(ones_v, acc_sh.at[di], sem).wait()
            return carry

        lax.fori_loop(0, nch, drain, 0)
        plsc.subcore_barrier()
        ooff = pl.multiple_of(c * n_pad + s * rpt, 128)
        pltpu.sync_copy(acc_sh.at[pl.ds(soff, rpt)],
                        out_hbm.at[pl.ds(ooff, rpt)])

    return deg_kernel(dst, zeros1)


_NBUF = 5
_LA = 4


def _sc_propagate(feat, src, dst, zrows):
    n = feat.shape[0]
    e = src.shape[0]
    epw = e // NW
    nch = epw // KCH
    assert nch % _NBUF == 0 and nch // _NBUF >= 3
    rpt = ((n + NS - 1) // NS + 7) // 8 * 8
    rlast = n - (NS - 1) * rpt
    mesh = plsc.VectorSubcoreMesh(core_axis_name="c", subcore_axis_name="s")

    @functools.partial(
        pl.kernel,
        mesh=mesh,
        out_type=jax.ShapeDtypeStruct((NC, n, 128), jnp.float32),
        scratch_types=[
            pltpu.VMEM((epw,), jnp.int32),
            pltpu.VMEM((epw,), jnp.int32),
            pltpu.VMEM((_NBUF, KCH, 128), jnp.float32),
            pltpu.VMEM_SHARED((n, 128), jnp.float32),
        ] + [pltpu.SemaphoreType.DMA] * (2 * _NBUF),
    )
    def prop_kernel(f_hbm, src_hbm, dst_hbm, z_hbm, out_hbm,
                    si_all, di_all, rows, acc_sh, *sems):
        sem_g = sems[:_NBUF]
        sem_s = sems[_NBUF:]
        c = lax.axis_index("c")
        s = lax.axis_index("s")
        wid = s * NC + c
        soff = pl.multiple_of(s * rpt, 8)
        ebase = pl.multiple_of(wid * epw, 8)

        pltpu.sync_copy(src_hbm.at[pl.ds(ebase, epw)], si_all)
        pltpu.sync_copy(dst_hbm.at[pl.ds(ebase, epw)], di_all)

        def seed(nrows):
            @pl.when(c == 0)
            def _():
                pltpu.sync_copy(f_hbm.at[pl.ds(soff, nrows)],
                                acc_sh.at[pl.ds(soff, nrows)])

            @pl.when(c != 0)
            def _():
                pltpu.sync_copy(z_hbm.at[pl.ds(soff, nrows)],
                                acc_sh.at[pl.ds(soff, nrows)])

        @pl.when(s < NS - 1)
        def _():
            seed(rpt)

        @pl.when(s == NS - 1)
        def _():
            seed(rlast)

        plsc.subcore_barrier()

        def start_gather(i, b):
            si = si_all[pl.ds(i * KCH, KCH)]
            pltpu.async_copy(f_hbm.at[si], rows.at[b], sem_g[b])

        def wait_gather(i, b):
            si = si_all[pl.ds(i * KCH, KCH)]
            pltpu.make_async_copy(f_hbm.at[si], rows.at[b], sem_g[b]).wait()

        def start_scatter(i, b):
            di = di_all[pl.ds(i * KCH, KCH)]
            pltpu.async_copy(rows.at[b], acc_sh.at[di], sem_s[b], add=True)

        def wait_scatter(i, b):
            di = di_all[pl.ds(i * KCH, KCH)]
            pltpu.make_async_copy(rows.at[b], acc_sh.at[di], sem_s[b]).wait()

        for i in range(_LA):
            start_gather(i, i)

        def step(i, b, bk, first):
            wait_gather(i, b)
            start_scatter(i, b)
            k = i + _LA
            if not first:
                wait_scatter(k - _NBUF, bk)
            start_gather(k, bk)

        for b in range(_NBUF):
            k = b + _LA
            if k < _NBUF:
                step(b, b, k, first=True)
            else:
                step(b, b, k - _NBUF, first=False)

        def group(jo, carry):
            j = jo * _NBUF
            for b in range(_NBUF):
                i = j + b
                step(i, b, (b + _LA) % _NBUF, first=False)
            return carry

        lax.fori_loop(1, nch // _NBUF - 1, group, 0)

        jl = nch - _NBUF
        for b in range(_NBUF):
            i = jl + b
            k = i + _LA
            wait_gather(i, b)
            start_scatter(i, b)
            if k < nch:
                bk = (b + _LA) % _NBUF
                wait_scatter(k - _NBUF, bk)
                start_gather(k, bk)

        for b in range(_NBUF):
            wait_scatter(jl + b, b)

        plsc.subcore_barrier()

        @pl.when(s < NS - 1)
        def _():
            pltpu.sync_copy(acc_sh.at[pl.ds(soff, rpt)],
                            out_hbm.at[c, pl.ds(soff, rpt)])

        @pl.when(s == NS - 1)
        def _():
            pltpu.sync_copy(acc_sh.at[pl.ds(soff, rlast)],
                            out_hbm.at[c, pl.ds(soff, rlast)])

    return prop_kernel(feat, src, dst, zrows)

    return prop_kernel(feat, src, dst, zrows)


_ROWS = 1000


def _tc_pre(degp, x, w0):
    n = x.shape[0]
    nb = n // _ROWS

    def body(degp_ref, x_ref, w_ref, f_ref, dinv_ref):
        dinv = lax.rsqrt(degp_ref[0] + degp_ref[1] + 1.0)
        dinv_ref[...] = dinv
        f_ref[...] = dinv * jnp.dot(x_ref[...], w_ref[...],
                                    preferred_element_type=jnp.float32)

    return pl.pallas_call(
        body,
        grid=(nb,),
        in_specs=[
            pl.BlockSpec((2, _ROWS, 1), lambda i: (0, i, 0)),
            pl.BlockSpec((_ROWS, 128), lambda i: (i, 0)),
            pl.BlockSpec((128, 128), lambda i: (0, 0)),
        ],
        out_specs=[
            pl.BlockSpec((_ROWS, 128), lambda i: (i, 0)),
            pl.BlockSpec((_ROWS, 1), lambda i: (i, 0)),
        ],
        out_shape=[
            jax.ShapeDtypeStruct((n, 128), jnp.float32),
            jax.ShapeDtypeStruct((n, 1), jnp.float32),
        ],
    )(degp, x, w0)


def _tc_mid(dinv, accp, gam, bet, wn):
    n = accp.shape[1]
    nb = n // _ROWS
    nf = float(n)

    def body(dinv_ref, acc_ref, gam_ref, bet_ref, w_ref, out_ref, stats):
        p = pl.program_id(0)
        z = dinv_ref[...] * (acc_ref[0] + acc_ref[1])

        @pl.when(p == 0)
        def _():
            @pl.when(pl.program_id(1) == 0)
            def _():
                stats[...] = jnp.zeros_like(stats)

            stats[0:1, :] = stats[0:1, :] + jnp.sum(z, axis=0, keepdims=True)
            stats[1:2, :] = stats[1:2, :] + jnp.sum(z * z, axis=0,
                                                    keepdims=True)

        @pl.when(p == 1)
        def _():
            m = stats[0:1, :] / nf
            var = stats[1:2, :] / nf - m * m
            rs = lax.rsqrt(var + 1e-5)
            h = jnp.maximum((z - m) * rs * gam_ref[...] + bet_ref[...], 0.0)
            out_ref[...] = dinv_ref[...] * jnp.dot(
                h, w_ref[...], preferred_element_type=jnp.float32)

    return pl.pallas_call(
        body,
        grid=(2, nb),
        in_specs=[
            pl.BlockSpec((_ROWS, 1), lambda p, i: (i, 0)),
            pl.BlockSpec((2, _ROWS, 128), lambda p, i: (0, i, 0)),
            pl.BlockSpec((1, 128), lambda p, i: (0, 0)),
            pl.BlockSpec((1, 128), lambda p, i: (0, 0)),
            pl.BlockSpec((128, 128), lambda p, i: (0, 0)),
        ],
        out_specs=pl.BlockSpec((_ROWS, 128), lambda p, i: (i, 0)),
        out_shape=jax.ShapeDtypeStruct((n, 128), jnp.float32),
        scratch_shapes=[pltpu.VMEM((2, 128), jnp.float32)],
    )(dinv, accp, gam, bet, wn)


def _tc_final(dinv, accp, b2, wfc, bfc):
    n = accp.shape[1]
    c = wfc.shape[1]
    nb = n // _ROWS

    def body(dinv_ref, acc_ref, b_ref, w_ref, bf_ref, out_ref):
        z = dinv_ref[...] * (acc_ref[0] + acc_ref[1]) + b_ref[...]
        out_ref[...] = jnp.dot(z, w_ref[...],
                               preferred_element_type=jnp.float32) + bf_ref[...]

    return pl.pallas_call(
        body,
        grid=(nb,),
        in_specs=[
            pl.BlockSpec((_ROWS, 1), lambda i: (i, 0)),
            pl.BlockSpec((2, _ROWS, 128), lambda i: (0, i, 0)),
            pl.BlockSpec((1, 128), lambda i: (0, 0)),
            pl.BlockSpec((128, c), lambda i: (0, 0)),
            pl.BlockSpec((1, c), lambda i: (0, 0)),
        ],
        out_specs=pl.BlockSpec((_ROWS, c), lambda i: (i, 0)),
        out_shape=jax.ShapeDtypeStruct((n, c), jnp.float32),
    )(dinv, accp, b2, wfc, bfc)


def kernel(x, edge_index, W0, b0, g0, beta0, W1, b1, g1, beta1, W2, b2,
           Wfc, bfc):
    n = x.shape[0]
    src = edge_index[0]
    dst = edge_index[1]
    n_pad = ((n + NS * 128 - 1) // (NS * 128)) * (NS * 128)
    zeros1 = jnp.zeros((n_pad,), jnp.float32)
    zrows = jnp.zeros((n, 128), jnp.float32)

    degp = _sc_degree(dst, zeros1, n_pad).reshape(2, n_pad)[:, :n]
    degp = degp.reshape(2, n, 1)
    feat0, dinv = _tc_pre(degp, x, W0)
    acc0 = _sc_propagate(feat0, src, dst, zrows)
    feat1 = _tc_mid(dinv, acc0, g0.reshape(1, 128), beta0.reshape(1, 128), W1)
    acc1 = _sc_propagate(feat1, src, dst, zrows)
    feat2 = _tc_mid(dinv, acc1, g1.reshape(1, 128), beta1.reshape(1, 128), W2)
    acc2 = _sc_propagate(feat2, src, dst, zrows)
    return _tc_final(dinv, acc2, b2.reshape(1, 128), Wfc,
                     bfc.reshape(1, bfc.shape[0]))


# speedup vs baseline: 27.2932x; 1.2063x over previous
import functools

import jax
import jax.numpy as jnp
from jax import lax
from jax.experimental import pallas as pl
from jax.experimental.pallas import tpu as pltpu
from jax.experimental.pallas import tpu_sc as plsc

NC = 2
NS = 16
NW = NC * NS
KCH = 16


def _sc_degree(dst, zeros1, n_pad):
    e = dst.shape[0]
    epw = e // NW
    nch = epw // KCH
    rpt = n_pad // NS
    mesh = plsc.VectorSubcoreMesh(core_axis_name="c", subcore_axis_name="s")

    @functools.partial(
        pl.kernel,
        mesh=mesh,
        out_type=jax.ShapeDtypeStruct((NC * n_pad,), jnp.float32),
        scratch_types=[
            pltpu.VMEM((epw,), jnp.int32),
            pltpu.VMEM((KCH,), jnp.float32),
            pltpu.VMEM_SHARED((n_pad,), jnp.float32),
            pltpu.SemaphoreType.DMA,
        ],
    )
    def deg_kernel(dst_hbm, z_hbm, out_hbm, di_all, ones_v, acc_sh, sem):
        c = lax.axis_index("c")
        s = lax.axis_index("s")
        wid = s * NC + c
        soff = pl.multiple_of(s * rpt, 128)
        ones_v[...] = jnp.full((KCH,), 1.0, dtype=jnp.float32)
        pltpu.sync_copy(
            dst_hbm.at[pl.ds(pl.multiple_of(wid * epw, 8), epw)], di_all)
        pltpu.sync_copy(z_hbm.at[pl.ds(soff, rpt)],
                        acc_sh.at[pl.ds(soff, rpt)])
        plsc.subcore_barrier()

        def fire(i, carry):
            di = di_all[pl.ds(i * KCH, KCH)]
            pltpu.async_copy(ones_v, acc_sh.at[di], sem, add=True)
            return carry

        lax.fori_loop(0, nch, fire, 0)

        def drain(i, carry):
            di = di_all[pl.ds(i * KCH, KCH)]
            pltpu.make_async_copy(ones_v, acc_sh.at[di], sem).wait()
            return carry

        lax.fori_loop(0, nch, drain, 0)
        plsc.subcore_barrier()
        ooff = pl.multiple_of(c * n_pad + s * rpt, 128)
        pltpu.sync_copy(acc_sh.at[pl.ds(soff, rpt)],
                        out_hbm.at[pl.ds(ooff, rpt)])

    return deg_kernel(dst, zeros1)


_NBUF = 8
_LA = 6


def _sc_propagate(feat, src, dst, zrows):
    n = feat.shape[0]
    e = src.shape[0]
    epw = e // NW
    nch = epw // KCH
    assert _LA <= _NBUF and nch >= 2 * _NBUF + _LA
    rpt = ((n + NS - 1) // NS + 7) // 8 * 8
    rlast = n - (NS - 1) * rpt
    mesh = plsc.VectorSubcoreMesh(core_axis_name="c", subcore_axis_name="s")

    @functools.partial(
        pl.kernel,
        mesh=mesh,
        out_type=jax.ShapeDtypeStruct((NC, n, 128), jnp.float32),
        scratch_types=[
            pltpu.VMEM((epw,), jnp.int32),
            pltpu.VMEM((epw,), jnp.int32),
            pltpu.VMEM((_NBUF, KCH, 128), jnp.float32),
            pltpu.VMEM_SHARED((n, 128), jnp.float32),
        ] + [pltpu.SemaphoreType.DMA] * (2 * _NBUF),
    )
    def prop_kernel(f_hbm, src_hbm, dst_hbm, z_hbm, out_hbm,
                    si_all, di_all, rows, acc_sh, *sems):
        sem_g = sems[:_NBUF]
        sem_s = sems[_NBUF:]
        c = lax.axis_index("c")
        s = lax.axis_index("s")
        wid = s * NC + c
        soff = pl.multiple_of(s * rpt, 8)
        ebase = pl.multiple_of(wid * epw, 8)

        pltpu.sync_copy(src_hbm.at[pl.ds(ebase, epw)], si_all)
        pltpu.sync_copy(dst_hbm.at[pl.ds(ebase, epw)], di_all)

        def seed(nrows):
            @pl.when(c == 0)
            def _():
                pltpu.sync_copy(f_hbm.at[pl.ds(soff, nrows)],
                                acc_sh.at[pl.ds(soff, nrows)])

            @pl.when(c != 0)
            def _():
                pltpu.sync_copy(z_hbm.at[pl.ds(soff, nrows)],
                                acc_sh.at[pl.ds(soff, nrows)])

        @pl.when(s < NS - 1)
        def _():
            seed(rpt)

        @pl.when(s == NS - 1)
        def _():
            seed(rlast)

        plsc.subcore_barrier()

        def start_gather(i, b):
            si = si_all[pl.ds(i * KCH, KCH)]
            pltpu.async_copy(f_hbm.at[si], rows.at[b], sem_g[b])

        def wait_gather(i, b):
            si = si_all[pl.ds(i * KCH, KCH)]
            pltpu.make_async_copy(f_hbm.at[si], rows.at[b], sem_g[b]).wait()

        def start_scatter(i, b):
            di = di_all[pl.ds(i * KCH, KCH)]
            pltpu.async_copy(rows.at[b], acc_sh.at[di], sem_s[b], add=True)

        def wait_scatter(i, b):
            di = di_all[pl.ds(i * KCH, KCH)]
            pltpu.make_async_copy(rows.at[b], acc_sh.at[di], sem_s[b]).wait()

        for i in range(_LA):
            start_gather(i, i)

        def step(i, b, do_swait, do_prefetch):
            wait_gather(i, b)
            start_scatter(i, b)
            if do_prefetch:
                k = i + _LA
                bk = (b + _LA) % _NBUF
                if do_swait:
                    wait_scatter(k - _NBUF, bk)
                start_gather(k, bk)

        for b in range(_NBUF):
            step(b, b, do_swait=(b + _LA >= _NBUF), do_prefetch=True)

        m = (nch - _LA) // _NBUF

        def group(jo, carry):
            j = jo * _NBUF
            for b in range(_NBUF):
                step(j + b, b, do_swait=True, do_prefetch=True)
            return carry

        lax.fori_loop(1, m, group, 0,
                      unroll=False) if m > 1 else None

        for t in range(m * _NBUF, nch):
            step(t, t % _NBUF, do_swait=True, do_prefetch=(t + _LA < nch))

        for i in range(nch - _NBUF, nch):
            wait_scatter(i, i % _NBUF)

        plsc.subcore_barrier()

        @pl.when(s < NS - 1)
        def _():
            pltpu.sync_copy(acc_sh.at[pl.ds(soff, rpt)],
                            out_hbm.at[c, pl.ds(soff, rpt)])

        @pl.when(s == NS - 1)
        def _():
            pltpu.sync_copy(acc_sh.at[pl.ds(soff, rlast)],
                            out_hbm.at[c, pl.ds(soff, rlast)])

    return prop_kernel(feat, src, dst, zrows)

    return prop_kernel(feat, src, dst, zrows)


_ROWS = 1000


def _tc_pre(degp, x, w0):
    n = x.shape[0]
    nb = n // _ROWS

    def body(degp_ref, x_ref, w_ref, f_ref, dinv_ref):
        dinv = lax.rsqrt(degp_ref[0] + degp_ref[1] + 1.0)
        dinv_ref[...] = dinv
        f_ref[...] = dinv * jnp.dot(x_ref[...], w_ref[...],
                                    preferred_element_type=jnp.float32)

    return pl.pallas_call(
        body,
        grid=(nb,),
        in_specs=[
            pl.BlockSpec((2, _ROWS, 1), lambda i: (0, i, 0)),
            pl.BlockSpec((_ROWS, 128), lambda i: (i, 0)),
            pl.BlockSpec((128, 128), lambda i: (0, 0)),
        ],
        out_specs=[
            pl.BlockSpec((_ROWS, 128), lambda i: (i, 0)),
            pl.BlockSpec((_ROWS, 1), lambda i: (i, 0)),
        ],
        out_shape=[
            jax.ShapeDtypeStruct((n, 128), jnp.float32),
            jax.ShapeDtypeStruct((n, 1), jnp.float32),
        ],
    )(degp, x, w0)


def _tc_mid(dinv, accp, gam, bet, wn):
    n = accp.shape[1]
    nb = n // _ROWS
    nf = float(n)

    def body(dinv_ref, acc_ref, gam_ref, bet_ref, w_ref, out_ref, stats):
        p = pl.program_id(0)
        z = dinv_ref[...] * (acc_ref[0] + acc_ref[1])

        @pl.when(p == 0)
        def _():
            @pl.when(pl.program_id(1) == 0)
            def _():
                stats[...] = jnp.zeros_like(stats)

            stats[0:1, :] = stats[0:1, :] + jnp.sum(z, axis=0, keepdims=True)
            stats[1:2, :] = stats[1:2, :] + jnp.sum(z * z, axis=0,
                                                    keepdims=True)

        @pl.when(p == 1)
        def _():
            m = stats[0:1, :] / nf
            var = stats[1:2, :] / nf - m * m
            rs = lax.rsqrt(var + 1e-5)
            h = jnp.maximum((z - m) * rs * gam_ref[...] + bet_ref[...], 0.0)
            out_ref[...] = dinv_ref[...] * jnp.dot(
                h, w_ref[...], preferred_element_type=jnp.float32)

    return pl.pallas_call(
        body,
        grid=(2, nb),
        in_specs=[
            pl.BlockSpec((_ROWS, 1), lambda p, i: (i, 0)),
            pl.BlockSpec((2, _ROWS, 128), lambda p, i: (0, i, 0)),
            pl.BlockSpec((1, 128), lambda p, i: (0, 0)),
            pl.BlockSpec((1, 128), lambda p, i: (0, 0)),
            pl.BlockSpec((128, 128), lambda p, i: (0, 0)),
        ],
        out_specs=pl.BlockSpec((_ROWS, 128), lambda p, i: (i, 0)),
        out_shape=jax.ShapeDtypeStruct((n, 128), jnp.float32),
        scratch_shapes=[pltpu.VMEM((2, 128), jnp.float32)],
    )(dinv, accp, gam, bet, wn)


def _tc_final(dinv, accp, b2, wfc, bfc):
    n = accp.shape[1]
    c = wfc.shape[1]
    nb = n // _ROWS

    def body(dinv_ref, acc_ref, b_ref, w_ref, bf_ref, out_ref):
        z = dinv_ref[...] * (acc_ref[0] + acc_ref[1]) + b_ref[...]
        out_ref[...] = jnp.dot(z, w_ref[...],
                               preferred_element_type=jnp.float32) + bf_ref[...]

    return pl.pallas_call(
        body,
        grid=(nb,),
        in_specs=[
            pl.BlockSpec((_ROWS, 1), lambda i: (i, 0)),
            pl.BlockSpec((2, _ROWS, 128), lambda i: (0, i, 0)),
            pl.BlockSpec((1, 128), lambda i: (0, 0)),
            pl.BlockSpec((128, c), lambda i: (0, 0)),
            pl.BlockSpec((1, c), lambda i: (0, 0)),
        ],
        out_specs=pl.BlockSpec((_ROWS, c), lambda i: (i, 0)),
        out_shape=jax.ShapeDtypeStruct((n, c), jnp.float32),
    )(dinv, accp, b2, wfc, bfc)


def kernel(x, edge_index, W0, b0, g0, beta0, W1, b1, g1, beta1, W2, b2,
           Wfc, bfc):
    n = x.shape[0]
    src = edge_index[0]
    dst = edge_index[1]
    n_pad = ((n + NS * 128 - 1) // (NS * 128)) * (NS * 128)
    zeros1 = jnp.zeros((n_pad,), jnp.float32)
    zrows = jnp.zeros((n, 128), jnp.float32)

    degp = _sc_degree(dst, zeros1, n_pad).reshape(2, n_pad)[:, :n]
    degp = degp.reshape(2, n, 1)
    feat0, dinv = _tc_pre(degp, x, W0)
    acc0 = _sc_propagate(feat0, src, dst, zrows)
    feat1 = _tc_mid(dinv, acc0, g0.reshape(1, 128), beta0.reshape(1, 128), W1)
    acc1 = _sc_propagate(feat1, src, dst, zrows)
    feat2 = _tc_mid(dinv, acc1, g1.reshape(1, 128), beta1.reshape(1, 128), W2)
    acc2 = _sc_propagate(feat2, src, dst, zrows)
    return _tc_final(dinv, acc2, b2.reshape(1, 128), Wfc,
                     bfc.reshape(1, bfc.shape[0]))


# speedup vs baseline: 28.6082x; 1.0482x over previous
import functools

import jax
import jax.numpy as jnp
from jax import lax
from jax.experimental import pallas as pl
from jax.experimental.pallas import tpu as pltpu
from jax.experimental.pallas import tpu_sc as plsc

NC = 2
NS = 16
NW = NC * NS
KCH = 16


def _sc_degree(dst, zeros1, n_pad):
    e = dst.shape[0]
    epw = e // NW
    nch = epw // KCH
    rpt = n_pad // NS
    mesh = plsc.VectorSubcoreMesh(core_axis_name="c", subcore_axis_name="s")

    @functools.partial(
        pl.kernel,
        mesh=mesh,
        out_type=jax.ShapeDtypeStruct((NC * n_pad,), jnp.float32),
        scratch_types=[
            pltpu.VMEM((epw,), jnp.int32),
            pltpu.VMEM((KCH,), jnp.float32),
            pltpu.VMEM_SHARED((n_pad,), jnp.float32),
            pltpu.SemaphoreType.DMA,
        ],
    )
    def deg_kernel(dst_hbm, z_hbm, out_hbm, di_all, ones_v, acc_sh, sem):
        c = lax.axis_index("c")
        s = lax.axis_index("s")
        wid = s * NC + c
        soff = pl.multiple_of(s * rpt, 128)
        ones_v[...] = jnp.full((KCH,), 1.0, dtype=jnp.float32)
        pltpu.sync_copy(
            dst_hbm.at[pl.ds(pl.multiple_of(wid * epw, 8), epw)], di_all)
        pltpu.sync_copy(z_hbm.at[pl.ds(soff, rpt)],
                        acc_sh.at[pl.ds(soff, rpt)])
        plsc.subcore_barrier()

        def fire(i, carry):
            di = di_all[pl.ds(i * KCH, KCH)]
            pltpu.async_copy(ones_v, acc_sh.at[di], sem, add=True)
            return carry

        lax.fori_loop(0, nch, fire, 0)

        def drain(i, carry):
            di = di_all[pl.ds(i * KCH, KCH)]
            pltpu.make_async_copy(ones_v, acc_sh.at[di], sem).wait()
            return carry

        lax.fori_loop(0, nch, drain, 0)
        plsc.subcore_barrier()
        ooff = pl.multiple_of(c * n_pad + s * rpt, 128)
        pltpu.sync_copy(acc_sh.at[pl.ds(soff, rpt)],
                        out_hbm.at[pl.ds(ooff, rpt)])

    return deg_kernel(dst, zeros1)


_NBUF = 9
_LA = 7


def _sc_propagate(feat, src, dst, zrows):
    n = feat.shape[0]
    e = src.shape[0]
    epw = e // NW
    nch = epw // KCH
    assert _LA <= _NBUF and nch >= 2 * _NBUF + _LA
    rpt = ((n + NS - 1) // NS + 7) // 8 * 8
    rlast = n - (NS - 1) * rpt
    mesh = plsc.VectorSubcoreMesh(core_axis_name="c", subcore_axis_name="s")

    @functools.partial(
        pl.kernel,
        mesh=mesh,
        out_type=jax.ShapeDtypeStruct((NC, n, 128), jnp.float32),
        scratch_types=[
            pltpu.VMEM((epw,), jnp.int32),
            pltpu.VMEM((epw,), jnp.int32),
            pltpu.VMEM((_NBUF, KCH, 128), jnp.float32),
            pltpu.VMEM_SHARED((n, 128), jnp.float32),
        ] + [pltpu.SemaphoreType.DMA] * (2 * _NBUF),
    )
    def prop_kernel(f_hbm, src_hbm, dst_hbm, z_hbm, out_hbm,
                    si_all, di_all, rows, acc_sh, *sems):
        sem_g = sems[:_NBUF]
        sem_s = sems[_NBUF:]
        c = lax.axis_index("c")
        s = lax.axis_index("s")
        wid = s * NC + c
        soff = pl.multiple_of(s * rpt, 8)
        ebase = pl.multiple_of(wid * epw, 8)

        pltpu.sync_copy(src_hbm.at[pl.ds(ebase, epw)], si_all)
        pltpu.sync_copy(dst_hbm.at[pl.ds(ebase, epw)], di_all)

        def seed(nrows):
            @pl.when(c == 0)
            def _():
                pltpu.sync_copy(f_hbm.at[pl.ds(soff, nrows)],
                                acc_sh.at[pl.ds(soff, nrows)])

            @pl.when(c != 0)
            def _():
                pltpu.sync_copy(z_hbm.at[pl.ds(soff, nrows)],
                                acc_sh.at[pl.ds(soff, nrows)])

        @pl.when(s < NS - 1)
        def _():
            seed(rpt)

        @pl.when(s == NS - 1)
        def _():
            seed(rlast)

        plsc.subcore_barrier()

        def start_gather(i, b):
            si = si_all[pl.ds(i * KCH, KCH)]
            pltpu.async_copy(f_hbm.at[si], rows.at[b], sem_g[b])

        def wait_gather(i, b):
            si = si_all[pl.ds(i * KCH, KCH)]
            pltpu.make_async_copy(f_hbm.at[si], rows.at[b], sem_g[b]).wait()

        def start_scatter(i, b):
            di = di_all[pl.ds(i * KCH, KCH)]
            pltpu.async_copy(rows.at[b], acc_sh.at[di], sem_s[b], add=True)

        def wait_scatter(i, b):
            di = di_all[pl.ds(i * KCH, KCH)]
            pltpu.make_async_copy(rows.at[b], acc_sh.at[di], sem_s[b]).wait()

        for i in range(_LA):
            start_gather(i, i)

        def step(i, b, do_swait, do_prefetch):
            wait_gather(i, b)
            start_scatter(i, b)
            if do_prefetch:
                k = i + _LA
                bk = (b + _LA) % _NBUF
                if do_swait:
                    wait_scatter(k - _NBUF, bk)
                start_gather(k, bk)

        for b in range(_NBUF):
            step(b, b, do_swait=(b + _LA >= _NBUF), do_prefetch=True)

        m = (nch - _LA) // _NBUF

        def group(jo, carry):
            j = jo * _NBUF
            for b in range(_NBUF):
                step(j + b, b, do_swait=True, do_prefetch=True)
            return carry

        lax.fori_loop(1, m, group, 0,
                      unroll=False) if m > 1 else None

        for t in range(m * _NBUF, nch):
            step(t, t % _NBUF, do_swait=True, do_prefetch=(t + _LA < nch))

        for i in range(nch - _NBUF, nch):
            wait_scatter(i, i % _NBUF)

        plsc.subcore_barrier()

        @pl.when(s < NS - 1)
        def _():
            pltpu.sync_copy(acc_sh.at[pl.ds(soff, rpt)],
                            out_hbm.at[c, pl.ds(soff, rpt)])

        @pl.when(s == NS - 1)
        def _():
            pltpu.sync_copy(acc_sh.at[pl.ds(soff, rlast)],
                            out_hbm.at[c, pl.ds(soff, rlast)])

    return prop_kernel(feat, src, dst, zrows)

    return prop_kernel(feat, src, dst, zrows)


_ROWS = 1000


def _tc_pre(degp, x, w0):
    n = x.shape[0]
    nb = n // _ROWS

    def body(degp_ref, x_ref, w_ref, f_ref, dinv_ref):
        dinv = lax.rsqrt(degp_ref[0] + degp_ref[1] + 1.0)
        dinv_ref[...] = dinv
        f_ref[...] = dinv * jnp.dot(x_ref[...], w_ref[...],
                                    preferred_element_type=jnp.float32)

    return pl.pallas_call(
        body,
        grid=(nb,),
        in_specs=[
            pl.BlockSpec((2, _ROWS, 1), lambda i: (0, i, 0)),
            pl.BlockSpec((_ROWS, 128), lambda i: (i, 0)),
            pl.BlockSpec((128, 128), lambda i: (0, 0)),
        ],
        out_specs=[
            pl.BlockSpec((_ROWS, 128), lambda i: (i, 0)),
            pl.BlockSpec((_ROWS, 1), lambda i: (i, 0)),
        ],
        out_shape=[
            jax.ShapeDtypeStruct((n, 128), jnp.float32),
            jax.ShapeDtypeStruct((n, 1), jnp.float32),
        ],
    )(degp, x, w0)


def _tc_mid(dinv, accp, gam, bet, wn):
    n = accp.shape[1]
    nb = n // _ROWS
    nf = float(n)

    def body(dinv_ref, acc_ref, gam_ref, bet_ref, w_ref, out_ref, stats):
        p = pl.program_id(0)
        z = dinv_ref[...] * (acc_ref[0] + acc_ref[1])

        @pl.when(p == 0)
        def _():
            @pl.when(pl.program_id(1) == 0)
            def _():
                stats[...] = jnp.zeros_like(stats)

            stats[0:1, :] = stats[0:1, :] + jnp.sum(z, axis=0, keepdims=True)
            stats[1:2, :] = stats[1:2, :] + jnp.sum(z * z, axis=0,
                                                    keepdims=True)

        @pl.when(p == 1)
        def _():
            m = stats[0:1, :] / nf
            var = stats[1:2, :] / nf - m * m
            rs = lax.rsqrt(var + 1e-5)
            h = jnp.maximum((z - m) * rs * gam_ref[...] + bet_ref[...], 0.0)
            out_ref[...] = dinv_ref[...] * jnp.dot(
                h, w_ref[...], preferred_element_type=jnp.float32)

    return pl.pallas_call(
        body,
        grid=(2, nb),
        in_specs=[
            pl.BlockSpec((_ROWS, 1), lambda p, i: (i, 0)),
            pl.BlockSpec((2, _ROWS, 128), lambda p, i: (0, i, 0)),
            pl.BlockSpec((1, 128), lambda p, i: (0, 0)),
            pl.BlockSpec((1, 128), lambda p, i: (0, 0)),
            pl.BlockSpec((128, 128), lambda p, i: (0, 0)),
        ],
        out_specs=pl.BlockSpec((_ROWS, 128), lambda p, i: (i, 0)),
        out_shape=jax.ShapeDtypeStruct((n, 128), jnp.float32),
        scratch_shapes=[pltpu.VMEM((2, 128), jnp.float32)],
    )(dinv, accp, gam, bet, wn)


def _tc_final(dinv, accp, b2, wfc, bfc):
    n = accp.shape[1]
    c = wfc.shape[1]
    nb = n // _ROWS

    def body(dinv_ref, acc_ref, b_ref, w_ref, bf_ref, out_ref):
        z = dinv_ref[...] * (acc_ref[0] + acc_ref[1]) + b_ref[...]
        out_ref[...] = jnp.dot(z, w_ref[...],
                               preferred_element_type=jnp.float32) + bf_ref[...]

    return pl.pallas_call(
        body,
        grid=(nb,),
        in_specs=[
            pl.BlockSpec((_ROWS, 1), lambda i: (i, 0)),
            pl.BlockSpec((2, _ROWS, 128), lambda i: (0, i, 0)),
            pl.BlockSpec((1, 128), lambda i: (0, 0)),
            pl.BlockSpec((128, c), lambda i: (0, 0)),
            pl.BlockSpec((1, c), lambda i: (0, 0)),
        ],
        out_specs=pl.BlockSpec((_ROWS, c), lambda i: (i, 0)),
        out_shape=jax.ShapeDtypeStruct((n, c), jnp.float32),
    )(dinv, accp, b2, wfc, bfc)


def kernel(x, edge_index, W0, b0, g0, beta0, W1, b1, g1, beta1, W2, b2,
           Wfc, bfc):
    n = x.shape[0]
    src = edge_index[0]
    dst = edge_index[1]
    n_pad = ((n + NS * 128 - 1) // (NS * 128)) * (NS * 128)
    zeros1 = jnp.zeros((n_pad,), jnp.float32)
    zrows = jnp.zeros((n, 128), jnp.float32)

    degp = _sc_degree(dst, zeros1, n_pad).reshape(2, n_pad)[:, :n]
    degp = degp.reshape(2, n, 1)
    feat0, dinv = _tc_pre(degp, x, W0)
    acc0 = _sc_propagate(feat0, src, dst, zrows)
    feat1 = _tc_mid(dinv, acc0, g0.reshape(1, 128), beta0.reshape(1, 128), W1)
    acc1 = _sc_propagate(feat1, src, dst, zrows)
    feat2 = _tc_mid(dinv, acc1, g1.reshape(1, 128), beta1.reshape(1, 128), W2)
    acc2 = _sc_propagate(feat2, src, dst, zrows)
    return _tc_final(dinv, acc2, b2.reshape(1, 128), Wfc,
                     bfc.reshape(1, bfc.shape[0]))


# speedup vs baseline: 29.8152x; 1.0422x over previous
import functools

import jax
import jax.numpy as jnp
from jax import lax
from jax.experimental import pallas as pl
from jax.experimental.pallas import tpu as pltpu
from jax.experimental.pallas import tpu_sc as plsc

NC = 2
NS = 16
NW = NC * NS
KCH = 16


def _sc_degree(edge_flat, zeros1, n_pad):
    e = edge_flat.shape[0] // 2
    epw = e // NW
    nch = epw // KCH
    rpt = n_pad // NS
    mesh = plsc.VectorSubcoreMesh(core_axis_name="c", subcore_axis_name="s")

    @functools.partial(
        pl.kernel,
        mesh=mesh,
        out_type=jax.ShapeDtypeStruct((NC * n_pad,), jnp.float32),
        scratch_types=[
            pltpu.VMEM((epw,), jnp.int32),
            pltpu.VMEM((KCH,), jnp.float32),
            pltpu.VMEM_SHARED((n_pad,), jnp.float32),
            pltpu.SemaphoreType.DMA,
        ],
    )
    def deg_kernel(dst_hbm, z_hbm, out_hbm, di_all, ones_v, acc_sh, sem):
        c = lax.axis_index("c")
        s = lax.axis_index("s")
        wid = s * NC + c
        soff = pl.multiple_of(s * rpt, 128)
        ones_v[...] = jnp.full((KCH,), 1.0, dtype=jnp.float32)
        pltpu.sync_copy(
            dst_hbm.at[pl.ds(pl.multiple_of(e + wid * epw, 8), epw)], di_all)
        pltpu.sync_copy(z_hbm.at[pl.ds(0, rpt)],
                        acc_sh.at[pl.ds(soff, rpt)])
        plsc.subcore_barrier()

        def fire(i, carry):
            di = di_all[pl.ds(i * KCH, KCH)]
            pltpu.async_copy(ones_v, acc_sh.at[di], sem, add=True)
            return carry

        lax.fori_loop(0, nch, fire, 0)

        def drain(i, carry):
            di = di_all[pl.ds(i * KCH, KCH)]
            pltpu.make_async_copy(ones_v, acc_sh.at[di], sem).wait()
            return carry

        lax.fori_loop(0, nch, drain, 0)
        plsc.subcore_barrier()
        ooff = pl.multiple_of(c * n_pad + s * rpt, 128)
        pltpu.sync_copy(acc_sh.at[pl.ds(soff, rpt)],
                        out_hbm.at[pl.ds(ooff, rpt)])

    return deg_kernel(edge_flat, zeros1)


_NBUF = 9
_LA = 7


def _sc_propagate(feat, edge_flat, zrows):
    n = feat.shape[0]
    e = edge_flat.shape[0] // 2
    epw = e // NW
    nch = epw // KCH
    assert _LA <= _NBUF and nch >= 2 * _NBUF + _LA
    rpt = ((n + NS - 1) // NS + 7) // 8 * 8
    rlast = n - (NS - 1) * rpt
    mesh = plsc.VectorSubcoreMesh(core_axis_name="c", subcore_axis_name="s")

    @functools.partial(
        pl.kernel,
        mesh=mesh,
        out_type=jax.ShapeDtypeStruct((NC, n, 128), jnp.float32),
        scratch_types=[
            pltpu.VMEM((epw,), jnp.int32),
            pltpu.VMEM((epw,), jnp.int32),
            pltpu.VMEM((_NBUF, KCH, 128), jnp.float32),
            pltpu.VMEM_SHARED((n, 128), jnp.float32),
        ] + [pltpu.SemaphoreType.DMA] * (2 * _NBUF),
    )
    def prop_kernel(f_hbm, edge_hbm, z_hbm, out_hbm,
                    si_all, di_all, rows, acc_sh, *sems):
        sem_g = sems[:_NBUF]
        sem_s = sems[_NBUF:]
        c = lax.axis_index("c")
        s = lax.axis_index("s")
        wid = s * NC + c
        soff = pl.multiple_of(s * rpt, 8)
        ebase = pl.multiple_of(wid * epw, 8)

        pltpu.sync_copy(edge_hbm.at[pl.ds(ebase, epw)], si_all)
        pltpu.sync_copy(edge_hbm.at[pl.ds(e + ebase, epw)], di_all)

        def seed(nrows):
            @pl.when(c == 0)
            def _():
                pltpu.sync_copy(f_hbm.at[pl.ds(soff, nrows)],
                                acc_sh.at[pl.ds(soff, nrows)])

            @pl.when(c != 0)
            def _():
                pltpu.sync_copy(z_hbm.at[pl.ds(0, nrows)],
                                acc_sh.at[pl.ds(soff, nrows)])

        @pl.when(s < NS - 1)
        def _():
            seed(rpt)

        @pl.when(s == NS - 1)
        def _():
            seed(rlast)

        plsc.subcore_barrier()

        def start_gather(i, b):
            si = si_all[pl.ds(i * KCH, KCH)]
            pltpu.async_copy(f_hbm.at[si], rows.at[b], sem_g[b])

        def wait_gather(i, b):
            si = si_all[pl.ds(i * KCH, KCH)]
            pltpu.make_async_copy(f_hbm.at[si], rows.at[b], sem_g[b]).wait()

        def start_scatter(i, b):
            di = di_all[pl.ds(i * KCH, KCH)]
            pltpu.async_copy(rows.at[b], acc_sh.at[di], sem_s[b], add=True)

        def wait_scatter(i, b):
            di = di_all[pl.ds(i * KCH, KCH)]
            pltpu.make_async_copy(rows.at[b], acc_sh.at[di], sem_s[b]).wait()

        for i in range(_LA):
            start_gather(i, i)

        def step(i, b, do_swait, do_prefetch):
            wait_gather(i, b)
            start_scatter(i, b)
            if do_prefetch:
                k = i + _LA
                bk = (b + _LA) % _NBUF
                if do_swait:
                    wait_scatter(k - _NBUF, bk)
                start_gather(k, bk)

        for b in range(_NBUF):
            step(b, b, do_swait=(b + _LA >= _NBUF), do_prefetch=True)

        m = (nch - _LA) // _NBUF

        def group(jo, carry):
            j = jo * _NBUF
            for b in range(_NBUF):
                step(j + b, b, do_swait=True, do_prefetch=True)
            return carry

        lax.fori_loop(1, m, group, 0,
                      unroll=False) if m > 1 else None

        for t in range(m * _NBUF, nch):
            step(t, t % _NBUF, do_swait=True, do_prefetch=(t + _LA < nch))

        for i in range(nch - _NBUF, nch):
            wait_scatter(i, i % _NBUF)

        plsc.subcore_barrier()

        @pl.when(s < NS - 1)
        def _():
            pltpu.sync_copy(acc_sh.at[pl.ds(soff, rpt)],
                            out_hbm.at[c, pl.ds(soff, rpt)])

        @pl.when(s == NS - 1)
        def _():
            pltpu.sync_copy(acc_sh.at[pl.ds(soff, rlast)],
                            out_hbm.at[c, pl.ds(soff, rlast)])

    return prop_kernel(feat, edge_flat, zrows)


_ROWS = 1000


def _tc_pre(degp, x, w0):
    n = x.shape[0]
    nb = n // _ROWS

    def body(degp_ref, x_ref, w_ref, f_ref, dinv_ref):
        dinv = lax.rsqrt(degp_ref[0] + degp_ref[1] + 1.0)
        dinv_ref[...] = dinv
        f_ref[...] = dinv * jnp.dot(x_ref[...], w_ref[...],
                                    preferred_element_type=jnp.float32)

    return pl.pallas_call(
        body,
        grid=(nb,),
        in_specs=[
            pl.BlockSpec((2, _ROWS, 1), lambda i: (0, i, 0)),
            pl.BlockSpec((_ROWS, 128), lambda i: (i, 0)),
            pl.BlockSpec((128, 128), lambda i: (0, 0)),
        ],
        out_specs=[
            pl.BlockSpec((_ROWS, 128), lambda i: (i, 0)),
            pl.BlockSpec((_ROWS, 1), lambda i: (i, 0)),
        ],
        out_shape=[
            jax.ShapeDtypeStruct((n, 128), jnp.float32),
            jax.ShapeDtypeStruct((n, 1), jnp.float32),
        ],
    )(degp, x, w0)


def _tc_mid(dinv, accp, gam, bet, wn):
    n = accp.shape[1]
    nb = n // _ROWS
    nf = float(n)

    def body(dinv_ref, acc_ref, gam_ref, bet_ref, w_ref, out_ref, stats,
             zbuf):
        p = pl.program_id(0)
        i = pl.program_id(1)

        @pl.when(p == 0)
        def _():
            z = dinv_ref[...] * (acc_ref[0] + acc_ref[1])
            zbuf[pl.ds(i * _ROWS, _ROWS), :] = z

            @pl.when(i == 0)
            def _():
                stats[...] = jnp.zeros_like(stats)

            stats[0:1, :] = stats[0:1, :] + jnp.sum(z, axis=0, keepdims=True)
            stats[1:2, :] = stats[1:2, :] + jnp.sum(z * z, axis=0,
                                                    keepdims=True)

        @pl.when(p == 1)
        def _():
            z = zbuf[pl.ds(i * _ROWS, _ROWS), :]
            m = stats[0:1, :] / nf
            var = stats[1:2, :] / nf - m * m
            rs = lax.rsqrt(var + 1e-5)
            h = jnp.maximum((z - m) * rs * gam_ref[...] + bet_ref[...], 0.0)
            out_ref[...] = dinv_ref[...] * jnp.dot(
                h, w_ref[...], preferred_element_type=jnp.float32)

    return pl.pallas_call(
        body,
        grid=(2, nb),
        in_specs=[
            pl.BlockSpec((_ROWS, 1), lambda p, i: (i, 0)),
            pl.BlockSpec((2, _ROWS, 128), lambda p, i: (0, i * (1 - p), 0)),
            pl.BlockSpec((1, 128), lambda p, i: (0, 0)),
            pl.BlockSpec((1, 128), lambda p, i: (0, 0)),
            pl.BlockSpec((128, 128), lambda p, i: (0, 0)),
        ],
        out_specs=pl.BlockSpec((_ROWS, 128), lambda p, i: (i, 0)),
        out_shape=jax.ShapeDtypeStruct((n, 128), jnp.float32),
        scratch_shapes=[pltpu.VMEM((2, 128), jnp.float32),
                        pltpu.VMEM((n, 128), jnp.float32)],
    )(dinv, accp, gam, bet, wn)


def _tc_final(dinv, accp, b2, wfc, bfc):
    n = accp.shape[1]
    c = wfc.shape[1]
    nb = n // _ROWS

    def body(dinv_ref, acc_ref, b_ref, w_ref, bf_ref, out_ref):
        z = dinv_ref[...] * (acc_ref[0] + acc_ref[1]) + b_ref[...]
        out_ref[...] = jnp.dot(z, w_ref[...],
                               preferred_element_type=jnp.float32) + bf_ref[...]

    return pl.pallas_call(
        body,
        grid=(nb,),
        in_specs=[
            pl.BlockSpec((_ROWS, 1), lambda i: (i, 0)),
            pl.BlockSpec((2, _ROWS, 128), lambda i: (0, i, 0)),
            pl.BlockSpec((1, 128), lambda i: (0, 0)),
            pl.BlockSpec((128, c), lambda i: (0, 0)),
            pl.BlockSpec((1, c), lambda i: (0, 0)),
        ],
        out_specs=pl.BlockSpec((_ROWS, c), lambda i: (i, 0)),
        out_shape=jax.ShapeDtypeStruct((n, c), jnp.float32),
    )(dinv, accp, b2, wfc, bfc)


def kernel(x, edge_index, W0, b0, g0, beta0, W1, b1, g1, beta1, W2, b2,
           Wfc, bfc):
    n = x.shape[0]
    e = edge_index.shape[1]
    edge_flat = edge_index.reshape(2 * e)
    n_pad = ((n + NS * 128 - 1) // (NS * 128)) * (NS * 128)
    rpt_pad = n_pad // NS
    rpt_n = ((n + NS - 1) // NS + 7) // 8 * 8
    zeros1 = jnp.zeros((rpt_pad,), jnp.float32)
    zrows = jnp.zeros((rpt_n, 128), jnp.float32)

    degp = _sc_degree(edge_flat, zeros1, n_pad).reshape(2, n_pad)[:, :n]
    degp = degp.reshape(2, n, 1)
    feat0, dinv = _tc_pre(degp, x, W0)
    acc0 = _sc_propagate(feat0, edge_flat, zrows)
    feat1 = _tc_mid(dinv, acc0, g0.reshape(1, 128), beta0.reshape(1, 128), W1)
    acc1 = _sc_propagate(feat1, edge_flat, zrows)
    feat2 = _tc_mid(dinv, acc1, g1.reshape(1, 128), beta1.reshape(1, 128), W2)
    acc2 = _sc_propagate(feat2, edge_flat, zrows)
    return _tc_final(dinv, acc2, b2.reshape(1, 128), Wfc,
                     bfc.reshape(1, bfc.shape[0]))


# speedup vs baseline: 31.8459x; 1.0681x over previous
import functools

import jax
import jax.numpy as jnp
from jax import lax
from jax.experimental import pallas as pl
from jax.experimental.pallas import tpu as pltpu
from jax.experimental.pallas import tpu_sc as plsc

NC = 2
NS = 16
NW = NC * NS
KCH = 16


def _sc_degree(edge_flat, zeros1, n_pad):
    e = edge_flat.shape[0] // 2
    epw = e // NW
    nch = epw // KCH
    rpt = n_pad // NS
    mesh = plsc.VectorSubcoreMesh(core_axis_name="c", subcore_axis_name="s")

    @functools.partial(
        pl.kernel,
        mesh=mesh,
        out_type=jax.ShapeDtypeStruct((NC * n_pad,), jnp.float32),
        scratch_types=[
            pltpu.VMEM((epw,), jnp.int32),
            pltpu.VMEM((KCH,), jnp.float32),
            pltpu.VMEM_SHARED((n_pad,), jnp.float32),
            pltpu.SemaphoreType.DMA,
        ],
    )
    def deg_kernel(dst_hbm, z_hbm, out_hbm, di_all, ones_v, acc_sh, sem):
        c = lax.axis_index("c")
        s = lax.axis_index("s")
        wid = s * NC + c
        soff = pl.multiple_of(s * rpt, 128)
        ones_v[...] = jnp.full((KCH,), 1.0, dtype=jnp.float32)
        pltpu.sync_copy(
            dst_hbm.at[pl.ds(pl.multiple_of(e + wid * epw, 8), epw)], di_all)
        pltpu.sync_copy(z_hbm.at[pl.ds(0, rpt)],
                        acc_sh.at[pl.ds(soff, rpt)])
        plsc.subcore_barrier()

        def fire(i, carry):
            di = di_all[pl.ds(i * KCH, KCH)]
            pltpu.async_copy(ones_v, acc_sh.at[di], sem, add=True)
            return carry

        lax.fori_loop(0, nch, fire, 0)

        def drain(i, carry):
            di = di_all[pl.ds(i * KCH, KCH)]
            pltpu.make_async_copy(ones_v, acc_sh.at[di], sem).wait()
            return carry

        lax.fori_loop(0, nch, drain, 0)
        plsc.subcore_barrier()
        ooff = pl.multiple_of(c * n_pad + s * rpt, 128)
        pltpu.sync_copy(acc_sh.at[pl.ds(soff, rpt)],
                        out_hbm.at[pl.ds(ooff, rpt)])

    return deg_kernel(edge_flat, zeros1)


_NBUF = 9
_LA = 8


def _sc_propagate(feat, edge_flat, zrows):
    n = feat.shape[0]
    e = edge_flat.shape[0] // 2
    epw = e // NW
    nch = epw // KCH
    assert _LA <= _NBUF and nch >= 2 * _NBUF + _LA
    rpt = ((n + NS - 1) // NS + 7) // 8 * 8
    rlast = n - (NS - 1) * rpt
    mesh = plsc.VectorSubcoreMesh(core_axis_name="c", subcore_axis_name="s")

    @functools.partial(
        pl.kernel,
        mesh=mesh,
        out_type=jax.ShapeDtypeStruct((NC, n, 128), jnp.float32),
        scratch_types=[
            pltpu.VMEM((epw,), jnp.int32),
            pltpu.VMEM((epw,), jnp.int32),
            pltpu.VMEM((_NBUF, KCH, 128), jnp.float32),
            pltpu.VMEM_SHARED((n, 128), jnp.float32),
        ] + [pltpu.SemaphoreType.DMA] * (2 * _NBUF),
    )
    def prop_kernel(f_hbm, edge_hbm, z_hbm, out_hbm,
                    si_all, di_all, rows, acc_sh, *sems):
        sem_g = sems[:_NBUF]
        sem_s = sems[_NBUF:]
        c = lax.axis_index("c")
        s = lax.axis_index("s")
        wid = s * NC + c
        soff = pl.multiple_of(s * rpt, 8)
        ebase = pl.multiple_of(wid * epw, 8)

        pltpu.sync_copy(edge_hbm.at[pl.ds(ebase, epw)], si_all)
        pltpu.sync_copy(edge_hbm.at[pl.ds(e + ebase, epw)], di_all)

        def seed(nrows):
            @pl.when(c == 0)
            def _():
                pltpu.sync_copy(f_hbm.at[pl.ds(soff, nrows)],
                                acc_sh.at[pl.ds(soff, nrows)])

            @pl.when(c != 0)
            def _():
                pltpu.sync_copy(z_hbm.at[pl.ds(0, nrows)],
                                acc_sh.at[pl.ds(soff, nrows)])

        @pl.when(s < NS - 1)
        def _():
            seed(rpt)

        @pl.when(s == NS - 1)
        def _():
            seed(rlast)

        plsc.subcore_barrier()

        def start_gather(i, b):
            si = si_all[pl.ds(i * KCH, KCH)]
            pltpu.async_copy(f_hbm.at[si], rows.at[b], sem_g[b])

        def wait_gather(i, b):
            si = si_all[pl.ds(i * KCH, KCH)]
            pltpu.make_async_copy(f_hbm.at[si], rows.at[b], sem_g[b]).wait()

        def start_scatter(i, b):
            di = di_all[pl.ds(i * KCH, KCH)]
            pltpu.async_copy(rows.at[b], acc_sh.at[di], sem_s[b], add=True)

        def wait_scatter(i, b):
            di = di_all[pl.ds(i * KCH, KCH)]
            pltpu.make_async_copy(rows.at[b], acc_sh.at[di], sem_s[b]).wait()

        for i in range(_LA):
            start_gather(i, i)

        def step(i, b, do_swait, do_prefetch):
            wait_gather(i, b)
            start_scatter(i, b)
            if do_prefetch:
                k = i + _LA
                bk = (b + _LA) % _NBUF
                if do_swait:
                    wait_scatter(k - _NBUF, bk)
                start_gather(k, bk)

        for b in range(_NBUF):
            step(b, b, do_swait=(b + _LA >= _NBUF), do_prefetch=True)

        m = (nch - _LA) // _NBUF

        def group(jo, carry):
            j = jo * _NBUF
            for b in range(_NBUF):
                step(j + b, b, do_swait=True, do_prefetch=True)
            return carry

        lax.fori_loop(1, m, group, 0,
                      unroll=False) if m > 1 else None

        for t in range(m * _NBUF, nch):
            step(t, t % _NBUF, do_swait=True, do_prefetch=(t + _LA < nch))

        for i in range(nch - _NBUF, nch):
            wait_scatter(i, i % _NBUF)

        plsc.subcore_barrier()

        @pl.when(s < NS - 1)
        def _():
            pltpu.sync_copy(acc_sh.at[pl.ds(soff, rpt)],
                            out_hbm.at[c, pl.ds(soff, rpt)])

        @pl.when(s == NS - 1)
        def _():
            pltpu.sync_copy(acc_sh.at[pl.ds(soff, rlast)],
                            out_hbm.at[c, pl.ds(soff, rlast)])

    return prop_kernel(feat, edge_flat, zrows)


_ROWS = 2000


def _tc_pre(degp, x, w0):
    n = x.shape[0]
    nb = n // _ROWS

    def body(degp_ref, x_ref, w_ref, f_ref, dinv_ref):
        dinv = lax.rsqrt(degp_ref[0] + degp_ref[1] + 1.0)
        dinv_ref[...] = dinv
        f_ref[...] = dinv * jnp.dot(x_ref[...], w_ref[...],
                                    preferred_element_type=jnp.float32)

    return pl.pallas_call(
        body,
        grid=(nb,),
        in_specs=[
            pl.BlockSpec((2, _ROWS, 1), lambda i: (0, i, 0)),
            pl.BlockSpec((_ROWS, 128), lambda i: (i, 0)),
            pl.BlockSpec((128, 128), lambda i: (0, 0)),
        ],
        out_specs=[
            pl.BlockSpec((_ROWS, 128), lambda i: (i, 0)),
            pl.BlockSpec((_ROWS, 1), lambda i: (i, 0)),
        ],
        out_shape=[
            jax.ShapeDtypeStruct((n, 128), jnp.float32),
            jax.ShapeDtypeStruct((n, 1), jnp.float32),
        ],
    )(degp, x, w0)


def _tc_mid(dinv, accp, gam, bet, wn):
    n = accp.shape[1]
    nb = n // _ROWS
    nf = float(n)

    def body(dinv_ref, acc_ref, gam_ref, bet_ref, w_ref, out_ref, stats,
             zbuf):
        p = pl.program_id(0)
        i = pl.program_id(1)

        @pl.when(p == 0)
        def _():
            z = dinv_ref[...] * (acc_ref[0] + acc_ref[1])
            zbuf[pl.ds(i * _ROWS, _ROWS), :] = z

            @pl.when(i == 0)
            def _():
                stats[...] = jnp.zeros_like(stats)

            stats[0:1, :] = stats[0:1, :] + jnp.sum(z, axis=0, keepdims=True)
            stats[1:2, :] = stats[1:2, :] + jnp.sum(z * z, axis=0,
                                                    keepdims=True)

        @pl.when(p == 1)
        def _():
            z = zbuf[pl.ds(i * _ROWS, _ROWS), :]
            m = stats[0:1, :] / nf
            var = stats[1:2, :] / nf - m * m
            rs = lax.rsqrt(var + 1e-5)
            h = jnp.maximum((z - m) * rs * gam_ref[...] + bet_ref[...], 0.0)
            out_ref[...] = dinv_ref[...] * jnp.dot(
                h, w_ref[...], preferred_element_type=jnp.float32)

    return pl.pallas_call(
        body,
        grid=(2, nb),
        in_specs=[
            pl.BlockSpec((_ROWS, 1), lambda p, i: (i, 0)),
            pl.BlockSpec((2, _ROWS, 128), lambda p, i: (0, i * (1 - p), 0)),
            pl.BlockSpec((1, 128), lambda p, i: (0, 0)),
            pl.BlockSpec((1, 128), lambda p, i: (0, 0)),
            pl.BlockSpec((128, 128), lambda p, i: (0, 0)),
        ],
        out_specs=pl.BlockSpec((_ROWS, 128), lambda p, i: (i, 0)),
        out_shape=jax.ShapeDtypeStruct((n, 128), jnp.float32),
        scratch_shapes=[pltpu.VMEM((2, 128), jnp.float32),
                        pltpu.VMEM((n, 128), jnp.float32)],
    )(dinv, accp, gam, bet, wn)


def _tc_final(dinv, accp, b2, wfc, bfc):
    n = accp.shape[1]
    c = wfc.shape[1]
    nb = n // _ROWS

    def body(dinv_ref, acc_ref, b_ref, w_ref, bf_ref, out_ref):
        z = dinv_ref[...] * (acc_ref[0] + acc_ref[1]) + b_ref[...]
        out_ref[...] = jnp.dot(z, w_ref[...],
                               preferred_element_type=jnp.float32) + bf_ref[...]

    return pl.pallas_call(
        body,
        grid=(nb,),
        in_specs=[
            pl.BlockSpec((_ROWS, 1), lambda i: (i, 0)),
            pl.BlockSpec((2, _ROWS, 128), lambda i: (0, i, 0)),
            pl.BlockSpec((1, 128), lambda i: (0, 0)),
            pl.BlockSpec((128, c), lambda i: (0, 0)),
            pl.BlockSpec((1, c), lambda i: (0, 0)),
        ],
        out_specs=pl.BlockSpec((_ROWS, c), lambda i: (i, 0)),
        out_shape=jax.ShapeDtypeStruct((n, c), jnp.float32),
    )(dinv, accp, b2, wfc, bfc)


def kernel(x, edge_index, W0, b0, g0, beta0, W1, b1, g1, beta1, W2, b2,
           Wfc, bfc):
    n = x.shape[0]
    e = edge_index.shape[1]
    edge_flat = edge_index.reshape(2 * e)
    n_pad = ((n + NS * 128 - 1) // (NS * 128)) * (NS * 128)
    rpt_pad = n_pad // NS
    rpt_n = ((n + NS - 1) // NS + 7) // 8 * 8
    zeros1 = jnp.zeros((rpt_pad,), jnp.float32)
    zrows = jnp.zeros((rpt_n, 128), jnp.float32)

    degp = _sc_degree(edge_flat, zeros1, n_pad).reshape(2, n_pad)[:, :n]
    degp = degp.reshape(2, n, 1)
    feat0, dinv = _tc_pre(degp, x, W0)
    acc0 = _sc_propagate(feat0, edge_flat, zrows)
    feat1 = _tc_mid(dinv, acc0, g0.reshape(1, 128), beta0.reshape(1, 128), W1)
    acc1 = _sc_propagate(feat1, edge_flat, zrows)
    feat2 = _tc_mid(dinv, acc1, g1.reshape(1, 128), beta1.reshape(1, 128), W2)
    acc2 = _sc_propagate(feat2, edge_flat, zrows)
    return _tc_final(dinv, acc2, b2.reshape(1, 128), Wfc,
                     bfc.reshape(1, bfc.shape[0]))


# speedup vs baseline: 32.1177x; 1.0085x over previous
import functools

import jax
import jax.numpy as jnp
from jax import lax
from jax.experimental import pallas as pl
from jax.experimental.pallas import tpu as pltpu
from jax.experimental.pallas import tpu_sc as plsc

NC = 2
NS = 16
NW = NC * NS
KCH = 16


def _sc_degree(edge_flat, zeros1, n_pad):
    e = edge_flat.shape[0] // 2
    epw = e // NW
    nch = epw // KCH
    rpt = n_pad // NS
    mesh = plsc.VectorSubcoreMesh(core_axis_name="c", subcore_axis_name="s")

    @functools.partial(
        pl.kernel,
        mesh=mesh,
        out_type=jax.ShapeDtypeStruct((NC * n_pad,), jnp.float32),
        scratch_types=[
            pltpu.VMEM((epw,), jnp.int32),
            pltpu.VMEM((KCH,), jnp.float32),
            pltpu.VMEM_SHARED((n_pad,), jnp.float32),
            pltpu.SemaphoreType.DMA,
        ],
    )
    def deg_kernel(dst_hbm, z_hbm, out_hbm, di_all, ones_v, acc_sh, sem):
        c = lax.axis_index("c")
        s = lax.axis_index("s")
        wid = s * NC + c
        soff = pl.multiple_of(s * rpt, 128)
        ones_v[...] = jnp.full((KCH,), 1.0, dtype=jnp.float32)
        pltpu.sync_copy(
            dst_hbm.at[pl.ds(pl.multiple_of(e + wid * epw, 8), epw)], di_all)
        pltpu.sync_copy(z_hbm.at[pl.ds(0, rpt)],
                        acc_sh.at[pl.ds(soff, rpt)])
        plsc.subcore_barrier()

        def fire(i, carry):
            di = di_all[pl.ds(i * KCH, KCH)]
            pltpu.async_copy(ones_v, acc_sh.at[di], sem, add=True)
            return carry

        lax.fori_loop(0, nch, fire, 0)

        def drain(i, carry):
            di = di_all[pl.ds(i * KCH, KCH)]
            pltpu.make_async_copy(ones_v, acc_sh.at[di], sem).wait()
            return carry

        lax.fori_loop(0, nch, drain, 0)
        plsc.subcore_barrier()
        ooff = pl.multiple_of(c * n_pad + s * rpt, 128)
        pltpu.sync_copy(acc_sh.at[pl.ds(soff, rpt)],
                        out_hbm.at[pl.ds(ooff, rpt)])

    return deg_kernel(edge_flat, zeros1)


_NBUF = 10
_LA = 8


def _sc_propagate(feat, edge_flat, zrows):
    n = feat.shape[0]
    e = edge_flat.shape[0] // 2
    epw = e // NW
    nch = epw // KCH
    assert _LA <= _NBUF and nch >= 2 * _NBUF + _LA
    rpt = ((n + NS - 1) // NS + 7) // 8 * 8
    rlast = n - (NS - 1) * rpt
    mesh = plsc.VectorSubcoreMesh(core_axis_name="c", subcore_axis_name="s")

    @functools.partial(
        pl.kernel,
        mesh=mesh,
        out_type=jax.ShapeDtypeStruct((NC, n, 128), jnp.float32),
        scratch_types=[
            pltpu.VMEM((epw,), jnp.int32),
            pltpu.VMEM((epw,), jnp.int32),
            pltpu.VMEM((_NBUF, KCH, 128), jnp.float32),
            pltpu.VMEM_SHARED((n, 128), jnp.float32),
        ] + [pltpu.SemaphoreType.DMA] * (2 * _NBUF),
    )
    def prop_kernel(f_hbm, edge_hbm, z_hbm, out_hbm,
                    si_all, di_all, rows, acc_sh, *sems):
        sem_g = sems[:_NBUF]
        sem_s = sems[_NBUF:]
        c = lax.axis_index("c")
        s = lax.axis_index("s")
        wid = s * NC + c
        soff = pl.multiple_of(s * rpt, 8)
        ebase = pl.multiple_of(wid * epw, 8)

        pltpu.sync_copy(edge_hbm.at[pl.ds(ebase, epw)], si_all)
        pltpu.sync_copy(edge_hbm.at[pl.ds(e + ebase, epw)], di_all)

        def seed(nrows):
            @pl.when(c == 0)
            def _():
                pltpu.sync_copy(f_hbm.at[pl.ds(soff, nrows)],
                                acc_sh.at[pl.ds(soff, nrows)])

            @pl.when(c != 0)
            def _():
                pltpu.sync_copy(z_hbm.at[pl.ds(0, nrows)],
                                acc_sh.at[pl.ds(soff, nrows)])

        @pl.when(s < NS - 1)
        def _():
            seed(rpt)

        @pl.when(s == NS - 1)
        def _():
            seed(rlast)

        plsc.subcore_barrier()

        def start_gather(i, b):
            si = si_all[pl.ds(i * KCH, KCH)]
            pltpu.async_copy(f_hbm.at[si], rows.at[b], sem_g[b])

        def wait_gather(i, b):
            si = si_all[pl.ds(i * KCH, KCH)]
            pltpu.make_async_copy(f_hbm.at[si], rows.at[b], sem_g[b]).wait()

        def start_scatter(i, b):
            di = di_all[pl.ds(i * KCH, KCH)]
            pltpu.async_copy(rows.at[b], acc_sh.at[di], sem_s[b], add=True)

        def wait_scatter(i, b):
            di = di_all[pl.ds(i * KCH, KCH)]
            pltpu.make_async_copy(rows.at[b], acc_sh.at[di], sem_s[b]).wait()

        for i in range(_LA):
            start_gather(i, i)

        def step(i, b, do_swait, do_prefetch):
            wait_gather(i, b)
            start_scatter(i, b)
            if do_prefetch:
                k = i + _LA
                bk = (b + _LA) % _NBUF
                if do_swait:
                    wait_scatter(k - _NBUF, bk)
                start_gather(k, bk)

        for b in range(_NBUF):
            step(b, b, do_swait=(b + _LA >= _NBUF), do_prefetch=True)

        m = (nch - _LA) // _NBUF

        def group(jo, carry):
            j = jo * _NBUF
            for b in range(_NBUF):
                step(j + b, b, do_swait=True, do_prefetch=True)
            return carry

        lax.fori_loop(1, m, group, 0,
                      unroll=False) if m > 1 else None

        for t in range(m * _NBUF, nch):
            step(t, t % _NBUF, do_swait=True, do_prefetch=(t + _LA < nch))

        for i in range(nch - _NBUF, nch):
            wait_scatter(i, i % _NBUF)

        plsc.subcore_barrier()

        @pl.when(s < NS - 1)
        def _():
            pltpu.sync_copy(acc_sh.at[pl.ds(soff, rpt)],
                            out_hbm.at[c, pl.ds(soff, rpt)])

        @pl.when(s == NS - 1)
        def _():
            pltpu.sync_copy(acc_sh.at[pl.ds(soff, rlast)],
                            out_hbm.at[c, pl.ds(soff, rlast)])

    return prop_kernel(feat, edge_flat, zrows)


_ROWS = 2000


def _tc_pre(degp, x, w0):
    n = x.shape[0]
    nb = n // _ROWS

    def body(degp_ref, x_ref, w_ref, f_ref, dinv_ref):
        dinv = lax.rsqrt(degp_ref[0] + degp_ref[1] + 1.0)
        dinv_ref[...] = dinv
        f_ref[...] = dinv * jnp.dot(x_ref[...], w_ref[...],
                                    preferred_element_type=jnp.float32)

    return pl.pallas_call(
        body,
        grid=(nb,),
        in_specs=[
            pl.BlockSpec((2, _ROWS, 1), lambda i: (0, i, 0)),
            pl.BlockSpec((_ROWS, 128), lambda i: (i, 0)),
            pl.BlockSpec((128, 128), lambda i: (0, 0)),
        ],
        out_specs=[
            pl.BlockSpec((_ROWS, 128), lambda i: (i, 0)),
            pl.BlockSpec((_ROWS, 1), lambda i: (i, 0)),
        ],
        out_shape=[
            jax.ShapeDtypeStruct((n, 128), jnp.float32),
            jax.ShapeDtypeStruct((n, 1), jnp.float32),
        ],
    )(degp, x, w0)


def _tc_mid(dinv, accp, gam, bet, wn):
    n = accp.shape[1]
    nb = n // _ROWS
    nf = float(n)

    def body(dinv_ref, acc_ref, gam_ref, bet_ref, w_ref, out_ref, stats,
             zbuf):
        p = pl.program_id(0)
        i = pl.program_id(1)

        @pl.when(p == 0)
        def _():
            z = dinv_ref[...] * (acc_ref[0] + acc_ref[1])
            zbuf[pl.ds(i * _ROWS, _ROWS), :] = z

            @pl.when(i == 0)
            def _():
                stats[...] = jnp.zeros_like(stats)

            stats[0:1, :] = stats[0:1, :] + jnp.sum(z, axis=0, keepdims=True)
            stats[1:2, :] = stats[1:2, :] + jnp.sum(z * z, axis=0,
                                                    keepdims=True)

        @pl.when(p == 1)
        def _():
            z = zbuf[pl.ds(i * _ROWS, _ROWS), :]
            m = stats[0:1, :] / nf
            var = stats[1:2, :] / nf - m * m
            rs = lax.rsqrt(var + 1e-5)
            h = jnp.maximum((z - m) * rs * gam_ref[...] + bet_ref[...], 0.0)
            out_ref[...] = dinv_ref[...] * jnp.dot(
                h, w_ref[...], preferred_element_type=jnp.float32)

    return pl.pallas_call(
        body,
        grid=(2, nb),
        in_specs=[
            pl.BlockSpec((_ROWS, 1), lambda p, i: (i, 0)),
            pl.BlockSpec((2, _ROWS, 128), lambda p, i: (0, i * (1 - p), 0)),
            pl.BlockSpec((1, 128), lambda p, i: (0, 0)),
            pl.BlockSpec((1, 128), lambda p, i: (0, 0)),
            pl.BlockSpec((128, 128), lambda p, i: (0, 0)),
        ],
        out_specs=pl.BlockSpec((_ROWS, 128), lambda p, i: (i, 0)),
        out_shape=jax.ShapeDtypeStruct((n, 128), jnp.float32),
        scratch_shapes=[pltpu.VMEM((2, 128), jnp.float32),
                        pltpu.VMEM((n, 128), jnp.float32)],
    )(dinv, accp, gam, bet, wn)


def _tc_final(dinv, accp, b2, wfc, bfc):
    n = accp.shape[1]
    c = wfc.shape[1]
    nb = n // _ROWS

    def body(dinv_ref, acc_ref, b_ref, w_ref, bf_ref, out_ref):
        z = dinv_ref[...] * (acc_ref[0] + acc_ref[1]) + b_ref[...]
        out_ref[...] = jnp.dot(z, w_ref[...],
                               preferred_element_type=jnp.float32) + bf_ref[...]

    return pl.pallas_call(
        body,
        grid=(nb,),
        in_specs=[
            pl.BlockSpec((_ROWS, 1), lambda i: (i, 0)),
            pl.BlockSpec((2, _ROWS, 128), lambda i: (0, i, 0)),
            pl.BlockSpec((1, 128), lambda i: (0, 0)),
            pl.BlockSpec((128, c), lambda i: (0, 0)),
            pl.BlockSpec((1, c), lambda i: (0, 0)),
        ],
        out_specs=pl.BlockSpec((_ROWS, c), lambda i: (i, 0)),
        out_shape=jax.ShapeDtypeStruct((n, c), jnp.float32),
    )(dinv, accp, b2, wfc, bfc)


def kernel(x, edge_index, W0, b0, g0, beta0, W1, b1, g1, beta1, W2, b2,
           Wfc, bfc):
    n = x.shape[0]
    e = edge_index.shape[1]
    edge_flat = edge_index.reshape(2 * e)
    n_pad = ((n + NS * 128 - 1) // (NS * 128)) * (NS * 128)
    rpt_pad = n_pad // NS
    rpt_n = ((n + NS - 1) // NS + 7) // 8 * 8
    zeros1 = jnp.zeros((rpt_pad,), jnp.float32)
    zrows = jnp.zeros((rpt_n, 128), jnp.float32)

    degp = _sc_degree(edge_flat, zeros1, n_pad).reshape(2, n_pad)[:, :n]
    degp = degp.reshape(2, n, 1)
    feat0, dinv = _tc_pre(degp, x, W0)
    acc0 = _sc_propagate(feat0, edge_flat, zrows)
    feat1 = _tc_mid(dinv, acc0, g0.reshape(1, 128), beta0.reshape(1, 128), W1)
    acc1 = _sc_propagate(feat1, edge_flat, zrows)
    feat2 = _tc_mid(dinv, acc1, g1.reshape(1, 128), beta1.reshape(1, 128), W2)
    acc2 = _sc_propagate(feat2, edge_flat, zrows)
    return _tc_final(dinv, acc2, b2.reshape(1, 128), Wfc,
                     bfc.reshape(1, bfc.shape[0]))


# speedup vs baseline: 32.7643x; 1.0201x over previous
import functools

import jax
import jax.numpy as jnp
from jax import lax
from jax.experimental import pallas as pl
from jax.experimental.pallas import tpu as pltpu
from jax.experimental.pallas import tpu_sc as plsc

NC = 2
NS = 16
NW = NC * NS
KCH = 16


def _sc_degree(edge_flat, zeros1, n_pad):
    e = edge_flat.shape[0] // 2
    epw = e // NW
    nch = epw // KCH
    rpt = n_pad // NS
    mesh = plsc.VectorSubcoreMesh(core_axis_name="c", subcore_axis_name="s")

    @functools.partial(
        pl.kernel,
        mesh=mesh,
        out_type=jax.ShapeDtypeStruct((NC * n_pad,), jnp.float32),
        scratch_types=[
            pltpu.VMEM((epw,), jnp.int32),
            pltpu.VMEM((KCH,), jnp.float32),
            pltpu.VMEM_SHARED((n_pad,), jnp.float32),
            pltpu.SemaphoreType.DMA,
        ],
    )
    def deg_kernel(dst_hbm, z_hbm, out_hbm, di_all, ones_v, acc_sh, sem):
        c = lax.axis_index("c")
        s = lax.axis_index("s")
        wid = s * NC + c
        soff = pl.multiple_of(s * rpt, 128)
        ones_v[...] = jnp.full((KCH,), 1.0, dtype=jnp.float32)
        pltpu.sync_copy(
            dst_hbm.at[pl.ds(pl.multiple_of(e + wid * epw, 8), epw)], di_all)
        pltpu.sync_copy(z_hbm.at[pl.ds(0, rpt)],
                        acc_sh.at[pl.ds(soff, rpt)])
        plsc.subcore_barrier()

        def fire(i, carry):
            di = di_all[pl.ds(i * KCH, KCH)]
            pltpu.async_copy(ones_v, acc_sh.at[di], sem, add=True)
            return carry

        lax.fori_loop(0, nch, fire, 0)

        def drain(i, carry):
            di = di_all[pl.ds(i * KCH, KCH)]
            pltpu.make_async_copy(ones_v, acc_sh.at[di], sem).wait()
            return carry

        lax.fori_loop(0, nch, drain, 0)
        plsc.subcore_barrier()
        ooff = pl.multiple_of(c * n_pad + s * rpt, 128)
        pltpu.sync_copy(acc_sh.at[pl.ds(soff, rpt)],
                        out_hbm.at[pl.ds(ooff, rpt)])

    return deg_kernel(edge_flat, zeros1)


_NBUF = 10
_LA = 9


def _sc_propagate(feat, edge_flat, zrows):
    n = feat.shape[0]
    e = edge_flat.shape[0] // 2
    epw = e // NW
    nch = epw // KCH
    assert _LA <= _NBUF and nch >= 2 * _NBUF + _LA
    rpt = ((n + NS - 1) // NS + 7) // 8 * 8
    rlast = n - (NS - 1) * rpt
    mesh = plsc.VectorSubcoreMesh(core_axis_name="c", subcore_axis_name="s")

    @functools.partial(
        pl.kernel,
        mesh=mesh,
        out_type=jax.ShapeDtypeStruct((NC, n, 128), jnp.float32),
        scratch_types=[
            pltpu.VMEM((epw,), jnp.int32),
            pltpu.VMEM((epw,), jnp.int32),
            pltpu.VMEM((_NBUF, KCH, 128), jnp.float32),
            pltpu.VMEM_SHARED((n, 128), jnp.float32),
        ] + [pltpu.SemaphoreType.DMA] * (2 * _NBUF),
    )
    def prop_kernel(f_hbm, edge_hbm, z_hbm, out_hbm,
                    si_all, di_all, rows, acc_sh, *sems):
        sem_g = sems[:_NBUF]
        sem_s = sems[_NBUF:]
        c = lax.axis_index("c")
        s = lax.axis_index("s")
        wid = s * NC + c
        soff = pl.multiple_of(s * rpt, 8)
        ebase = pl.multiple_of(wid * epw, 8)

        pltpu.sync_copy(edge_hbm.at[pl.ds(ebase, epw)], si_all)
        pltpu.sync_copy(edge_hbm.at[pl.ds(e + ebase, epw)], di_all)

        def seed(nrows):
            @pl.when(c == 0)
            def _():
                pltpu.sync_copy(f_hbm.at[pl.ds(soff, nrows)],
                                acc_sh.at[pl.ds(soff, nrows)])

            @pl.when(c != 0)
            def _():
                pltpu.sync_copy(z_hbm.at[pl.ds(0, nrows)],
                                acc_sh.at[pl.ds(soff, nrows)])

        @pl.when(s < NS - 1)
        def _():
            seed(rpt)

        @pl.when(s == NS - 1)
        def _():
            seed(rlast)

        plsc.subcore_barrier()

        def start_gather(i, b):
            si = si_all[pl.ds(i * KCH, KCH)]
            pltpu.async_copy(f_hbm.at[si], rows.at[b], sem_g[b])

        def wait_gather(i, b):
            si = si_all[pl.ds(i * KCH, KCH)]
            pltpu.make_async_copy(f_hbm.at[si], rows.at[b], sem_g[b]).wait()

        def start_scatter(i, b):
            di = di_all[pl.ds(i * KCH, KCH)]
            pltpu.async_copy(rows.at[b], acc_sh.at[di], sem_s[b], add=True)

        def wait_scatter(i, b):
            di = di_all[pl.ds(i * KCH, KCH)]
            pltpu.make_async_copy(rows.at[b], acc_sh.at[di], sem_s[b]).wait()

        for i in range(_LA):
            start_gather(i, i)

        def step(i, b, do_swait, do_prefetch):
            wait_gather(i, b)
            start_scatter(i, b)
            if do_prefetch:
                k = i + _LA
                bk = (b + _LA) % _NBUF
                if do_swait:
                    wait_scatter(k - _NBUF, bk)
                start_gather(k, bk)

        for b in range(_NBUF):
            step(b, b, do_swait=(b + _LA >= _NBUF), do_prefetch=True)

        m = (nch - _LA) // _NBUF

        def group(jo, carry):
            j = jo * _NBUF
            for b in range(_NBUF):
                step(j + b, b, do_swait=True, do_prefetch=True)
            return carry

        lax.fori_loop(1, m, group, 0,
                      unroll=False) if m > 1 else None

        for t in range(m * _NBUF, nch):
            step(t, t % _NBUF, do_swait=True, do_prefetch=(t + _LA < nch))

        for i in range(nch - _NBUF, nch):
            wait_scatter(i, i % _NBUF)

        plsc.subcore_barrier()

        @pl.when(s < NS - 1)
        def _():
            pltpu.sync_copy(acc_sh.at[pl.ds(soff, rpt)],
                            out_hbm.at[c, pl.ds(soff, rpt)])

        @pl.when(s == NS - 1)
        def _():
            pltpu.sync_copy(acc_sh.at[pl.ds(soff, rlast)],
                            out_hbm.at[c, pl.ds(soff, rlast)])

    return prop_kernel(feat, edge_flat, zrows)


_ROWS = 2000


def _tc_pre(degp, x, w0):
    n = x.shape[0]
    nb = n // _ROWS

    def body(degp_ref, x_ref, w_ref, f_ref, dinv_ref):
        dinv = lax.rsqrt(degp_ref[0] + degp_ref[1] + 1.0)
        dinv_ref[...] = dinv
        f_ref[...] = dinv * jnp.dot(x_ref[...], w_ref[...],
                                    preferred_element_type=jnp.float32)

    return pl.pallas_call(
        body,
        grid=(nb,),
        in_specs=[
            pl.BlockSpec((2, _ROWS, 1), lambda i: (0, i, 0)),
            pl.BlockSpec((_ROWS, 128), lambda i: (i, 0)),
            pl.BlockSpec((128, 128), lambda i: (0, 0)),
        ],
        out_specs=[
            pl.BlockSpec((_ROWS, 128), lambda i: (i, 0)),
            pl.BlockSpec((_ROWS, 1), lambda i: (i, 0)),
        ],
        out_shape=[
            jax.ShapeDtypeStruct((n, 128), jnp.float32),
            jax.ShapeDtypeStruct((n, 1), jnp.float32),
        ],
    )(degp, x, w0)


def _tc_mid(dinv, accp, gam, bet, wn):
    n = accp.shape[1]
    nb = n // _ROWS
    nf = float(n)

    def body(dinv_ref, acc_ref, gam_ref, bet_ref, w_ref, out_ref, stats,
             zbuf):
        p = pl.program_id(0)
        i = pl.program_id(1)

        @pl.when(p == 0)
        def _():
            z = dinv_ref[...] * (acc_ref[0] + acc_ref[1])
            zbuf[pl.ds(i * _ROWS, _ROWS), :] = z

            @pl.when(i == 0)
            def _():
                stats[...] = jnp.zeros_like(stats)

            stats[0:1, :] = stats[0:1, :] + jnp.sum(z, axis=0, keepdims=True)
            stats[1:2, :] = stats[1:2, :] + jnp.sum(z * z, axis=0,
                                                    keepdims=True)

        @pl.when(p == 1)
        def _():
            z = zbuf[pl.ds(i * _ROWS, _ROWS), :]
            m = stats[0:1, :] / nf
            var = stats[1:2, :] / nf - m * m
            rs = lax.rsqrt(var + 1e-5)
            h = jnp.maximum((z - m) * rs * gam_ref[...] + bet_ref[...], 0.0)
            out_ref[...] = dinv_ref[...] * jnp.dot(
                h, w_ref[...], preferred_element_type=jnp.float32)

    return pl.pallas_call(
        body,
        grid=(2, nb),
        in_specs=[
            pl.BlockSpec((_ROWS, 1), lambda p, i: (i, 0)),
            pl.BlockSpec((2, _ROWS, 128), lambda p, i: (0, i * (1 - p), 0)),
            pl.BlockSpec((1, 128), lambda p, i: (0, 0)),
            pl.BlockSpec((1, 128), lambda p, i: (0, 0)),
            pl.BlockSpec((128, 128), lambda p, i: (0, 0)),
        ],
        out_specs=pl.BlockSpec((_ROWS, 128), lambda p, i: (i, 0)),
        out_shape=jax.ShapeDtypeStruct((n, 128), jnp.float32),
        scratch_shapes=[pltpu.VMEM((2, 128), jnp.float32),
                        pltpu.VMEM((n, 128), jnp.float32)],
    )(dinv, accp, gam, bet, wn)


def _tc_final(dinv, accp, b2, wfc, bfc):
    n = accp.shape[1]
    c = wfc.shape[1]
    nb = n // _ROWS

    def body(dinv_ref, acc_ref, b_ref, w_ref, bf_ref, out_ref):
        z = dinv_ref[...] * (acc_ref[0] + acc_ref[1]) + b_ref[...]
        out_ref[...] = jnp.dot(z, w_ref[...],
                               preferred_element_type=jnp.float32) + bf_ref[...]

    return pl.pallas_call(
        body,
        grid=(nb,),
        in_specs=[
            pl.BlockSpec((_ROWS, 1), lambda i: (i, 0)),
            pl.BlockSpec((2, _ROWS, 128), lambda i: (0, i, 0)),
            pl.BlockSpec((1, 128), lambda i: (0, 0)),
            pl.BlockSpec((128, c), lambda i: (0, 0)),
            pl.BlockSpec((1, c), lambda i: (0, 0)),
        ],
        out_specs=pl.BlockSpec((_ROWS, c), lambda i: (i, 0)),
        out_shape=jax.ShapeDtypeStruct((n, c), jnp.float32),
    )(dinv, accp, b2, wfc, bfc)


def kernel(x, edge_index, W0, b0, g0, beta0, W1, b1, g1, beta1, W2, b2,
           Wfc, bfc):
    n = x.shape[0]
    e = edge_index.shape[1]
    edge_flat = edge_index.reshape(2 * e)
    n_pad = ((n + NS * 128 - 1) // (NS * 128)) * (NS * 128)
    rpt_pad = n_pad // NS
    rpt_n = ((n + NS - 1) // NS + 7) // 8 * 8
    zeros1 = jnp.zeros((rpt_pad,), jnp.float32)
    zrows = jnp.zeros((rpt_n, 128), jnp.float32)

    degp = _sc_degree(edge_flat, zeros1, n_pad).reshape(2, n_pad)[:, :n]
    degp = degp.reshape(2, n, 1)
    feat0, dinv = _tc_pre(degp, x, W0)
    acc0 = _sc_propagate(feat0, edge_flat, zrows)
    feat1 = _tc_mid(dinv, acc0, g0.reshape(1, 128), beta0.reshape(1, 128), W1)
    acc1 = _sc_propagate(feat1, edge_flat, zrows)
    feat2 = _tc_mid(dinv, acc1, g1.reshape(1, 128), beta1.reshape(1, 128), W2)
    acc2 = _sc_propagate(feat2, edge_flat, zrows)
    return _tc_final(dinv, acc2, b2.reshape(1, 128), Wfc,
                     bfc.reshape(1, bfc.shape[0]))
